# Initial kernel scaffold; baseline (speedup 1.0000x reference)
import functools

import jax
import jax.numpy as jnp
from jax import lax
from jax.experimental import pallas as pl
from jax.experimental.pallas import tpu as pltpu
from jax.experimental.pallas import tpu_sc as plsc

N = 10000
IN_DIM = 256

NR = 10240
NC = 2
NS = 16
EB = 128
RPS = NR // NS
BR = 512


def _agg_body(C, F, NBt, table, srcb, dstb, zeros_in, out,
              src_v, dst_v, buf_a, buf_b, zbuf, acc, sem_a, sem_b):
    cid = lax.axis_index("c")
    sid = lax.axis_index("s")
    rlo = sid * RPS
    cpc = C // NC
    pltpu.sync_copy(dstb.at[sid], dst_v)
    pltpu.sync_copy(zeros_in, zbuf)
    for local in range(cpc):
        chunk = cid * cpc + local
        pltpu.sync_copy(srcb.at[chunk, sid], src_v)
        for z in range(RPS // EB):
            pltpu.sync_copy(zbuf, acc.at[pl.ds(rlo + z * EB, EB)])
        plsc.subcore_barrier()
        pltpu.async_copy(table.at[src_v.at[0]], buf_a, sem_a)
        pltpu.async_copy(table.at[src_v.at[1]], buf_b, sem_b)

        def pair(p, carry):
            j0 = 2 * p
            pltpu.make_async_copy(table.at[src_v.at[j0]], buf_a, sem_a).wait()
            pltpu.sync_copy(buf_a, acc.at[dst_v.at[j0]], add=True)
            pltpu.async_copy(table.at[src_v.at[j0 + 2]], buf_a, sem_a)
            pltpu.make_async_copy(table.at[src_v.at[j0 + 1]], buf_b, sem_b).wait()
            pltpu.sync_copy(buf_b, acc.at[dst_v.at[j0 + 1]], add=True)
            pltpu.async_copy(table.at[src_v.at[j0 + 3]], buf_b, sem_b)
            return carry

        lax.fori_loop(0, NBt // 2 - 1, pair, 0)
        pltpu.make_async_copy(table.at[src_v.at[NBt - 2]], buf_a, sem_a).wait()
        pltpu.sync_copy(buf_a, acc.at[dst_v.at[NBt - 2]], add=True)
        pltpu.make_async_copy(table.at[src_v.at[NBt - 1]], buf_b, sem_b).wait()
        pltpu.sync_copy(buf_b, acc.at[dst_v.at[NBt - 1]], add=True)
        plsc.subcore_barrier()
        pltpu.sync_copy(acc.at[pl.ds(rlo, RPS)],
                        out.at[chunk, pl.ds(rlo, RPS)])


@functools.lru_cache(maxsize=None)
def _make_agg(C, F, NBt):
    mesh = plsc.VectorSubcoreMesh(core_axis_name="c", subcore_axis_name="s")
    return pl.kernel(
        functools.partial(_agg_body, C, F, NBt),
        out_type=jax.ShapeDtypeStruct((C, NR, F), jnp.float32),
        mesh=mesh,
        scratch_types=[
            pltpu.VMEM((NBt, EB), jnp.int32),
            pltpu.VMEM((NBt, EB), jnp.int32),
            pltpu.VMEM((EB, F), jnp.float32),
            pltpu.VMEM((EB, F), jnp.float32),
            pltpu.VMEM((EB, F), jnp.float32),
            pltpu.VMEM_SHARED((NR, F), jnp.float32),
            pltpu.SemaphoreType.DMA,
            pltpu.SemaphoreType.DMA,
        ],
    )


def _deg_body(NBt, dstb, ones_in, zeros_in, out, dst_v, ones_v, zbuf, acc):
    cid = lax.axis_index("c")
    sid = lax.axis_index("s")
    rlo = sid * RPS
    half = NBt // 2
    pltpu.sync_copy(dstb.at[sid], dst_v)
    pltpu.sync_copy(ones_in, ones_v)
    pltpu.sync_copy(zeros_in, zbuf)
    for z in range(RPS // EB):
        pltpu.sync_copy(zbuf, acc.at[pl.ds(rlo + z * EB, EB)])
    plsc.subcore_barrier()

    def body(j, carry):
        pltpu.sync_copy(ones_v, acc.at[dst_v.at[cid * half + j]], add=True)
        return carry

    lax.fori_loop(0, half, body, 0)
    plsc.subcore_barrier()
    pltpu.sync_copy(acc.at[pl.ds(rlo, RPS)], out.at[cid, pl.ds(rlo, RPS)])


@functools.lru_cache(maxsize=None)
def _make_deg(NBt):
    mesh = plsc.VectorSubcoreMesh(core_axis_name="c", subcore_axis_name="s")
    return pl.kernel(
        functools.partial(_deg_body, NBt),
        out_type=jax.ShapeDtypeStruct((NC, NR, 16), jnp.float32),
        mesh=mesh,
        scratch_types=[
            pltpu.VMEM((NBt, EB), jnp.int32),
            pltpu.VMEM((EB, 16), jnp.float32),
            pltpu.VMEM((EB, 16), jnp.float32),
            pltpu.VMEM_SHARED((NR, 16), jnp.float32),
        ],
    )


def _mm_first(x_pad, dinv, W, C_out, F_out):
    d_in = x_pad.shape[1]
    d_out = W.shape[1]

    def body(x_ref, dv_ref, w_ref, out_ref):
        res = jnp.dot(x_ref[...], w_ref[...],
                      preferred_element_type=jnp.float32) * dv_ref[...]
        for c2 in range(C_out):
            out_ref[c2] = res[:, c2 * F_out:(c2 + 1) * F_out]

    return pl.pallas_call(
        body,
        grid=(NR // BR,),
        in_specs=[
            pl.BlockSpec((BR, d_in), lambda i: (i, 0)),
            pl.BlockSpec((BR, 1), lambda i: (i, 0)),
            pl.BlockSpec((d_in, d_out), lambda i: (0, 0)),
        ],
        out_specs=pl.BlockSpec((C_out, BR, F_out), lambda i: (0, i, 0)),
        out_shape=jax.ShapeDtypeStruct((C_out, NR, F_out), jnp.float32),
    )(x_pad, dinv, W)


def _mm_mid(agg, dinv, b_prev, W, C_in, F_in, C_out, F_out):
    d_out = W.shape[1]
    w_r = W.reshape(C_in, F_in, d_out)
    b_r = b_prev.reshape(C_in, 1, F_in)

    def body(a_ref, dv_ref, b_ref, w_ref, out_ref):
        dv = dv_ref[...]
        acc = jnp.zeros((BR, d_out), jnp.float32)
        for c in range(C_in):
            xc = jnp.maximum(a_ref[c] * dv + b_ref[c], 0.0)
            acc = acc + jnp.dot(xc, w_ref[c], preferred_element_type=jnp.float32)
        res = acc * dv
        for c2 in range(C_out):
            out_ref[c2] = res[:, c2 * F_out:(c2 + 1) * F_out]

    return pl.pallas_call(
        body,
        grid=(NR // BR,),
        in_specs=[
            pl.BlockSpec((C_in, BR, F_in), lambda i: (0, i, 0)),
            pl.BlockSpec((BR, 1), lambda i: (i, 0)),
            pl.BlockSpec((C_in, 1, F_in), lambda i: (0, 0, 0)),
            pl.BlockSpec((C_in, F_in, d_out), lambda i: (0, 0, 0)),
        ],
        out_specs=pl.BlockSpec((C_out, BR, F_out), lambda i: (0, i, 0)),
        out_shape=jax.ShapeDtypeStruct((C_out, NR, F_out), jnp.float32),
    )(agg, dinv, b_r, w_r)


def _mm_last(agg, dinv, b4):
    b_r = b4.reshape(2, 1, 128)

    def body(a_ref, dv_ref, b_ref, out_ref):
        dv = dv_ref[...]
        for c in range(2):
            out_ref[:, c * 128:(c + 1) * 128] = a_ref[c] * dv + b_ref[c]

    return pl.pallas_call(
        body,
        grid=(NR // BR,),
        in_specs=[
            pl.BlockSpec((2, BR, 128), lambda i: (0, i, 0)),
            pl.BlockSpec((BR, 1), lambda i: (i, 0)),
            pl.BlockSpec((2, 1, 128), lambda i: (0, 0, 0)),
        ],
        out_specs=pl.BlockSpec((BR, 256), lambda i: (i, 0)),
        out_shape=jax.ShapeDtypeStruct((NR, 256), jnp.float32),
    )(agg, dinv, b_r)


def kernel(x, edge_index, W1, b1, W2, b2, W3, b3, W4, b4):
    E = edge_index.shape[1]
    loop = jnp.arange(N, dtype=jnp.int32)
    src_f = jnp.concatenate([edge_index[0], loop])
    dst_f = jnp.concatenate([edge_index[1], loop])
    e_full = E + N
    nbt = -(-e_full // (NS * EB)) + 2
    if nbt % 2:
        nbt += 1
    pad = NS * nbt * EB - e_full
    src_p = jnp.concatenate(
        [src_f, jnp.full((pad,), N, jnp.int32)]).reshape(NS, nbt, EB)
    dst_p = jnp.concatenate(
        [dst_f, jnp.full((pad,), N, jnp.int32)]).reshape(NS, nbt, EB)

    def chunked_src(C):
        off = (jnp.arange(C, dtype=jnp.int32) * NR).reshape(C, 1, 1, 1)
        return src_p[None] + off

    zeros128 = jnp.zeros((EB, 128), jnp.float32)
    zeros64 = jnp.zeros((EB, 64), jnp.float32)
    zeros16 = jnp.zeros((EB, 16), jnp.float32)
    ones16 = jnp.ones((EB, 16), jnp.float32)

    degs = _make_deg(nbt)(dst_p, ones16, zeros16)
    deg = degs[0, :, 0] + degs[1, :, 0]
    valid = (jnp.arange(NR) < N) & (deg > 0)
    dinv = jnp.where(valid, 1.0 / jnp.sqrt(jnp.maximum(deg, 1.0)), 0.0)
    dinv = dinv.reshape(NR, 1).astype(jnp.float32)

    x_pad = jnp.concatenate(
        [x, jnp.zeros((NR - N, IN_DIM), jnp.float32)], axis=0)

    t1 = _mm_first(x_pad, dinv, W1, 4, 128)
    a1 = _make_agg(4, 128, nbt)(
        t1.reshape(4 * NR, 128), chunked_src(4), dst_p, zeros128)
    t2 = _mm_mid(a1, dinv, b1, W2, 4, 128, 2, 64)
    a2 = _make_agg(2, 64, nbt)(
        t2.reshape(2 * NR, 64), chunked_src(2), dst_p, zeros64)
    t3 = _mm_mid(a2, dinv, b2, W3, 2, 64, 4, 128)
    a3 = _make_agg(4, 128, nbt)(
        t3.reshape(4 * NR, 128), chunked_src(4), dst_p, zeros128)
    t4 = _mm_mid(a3, dinv, b3, W4, 4, 128, 2, 128)
    a4 = _make_agg(2, 128, nbt)(
        t4.reshape(2 * NR, 128), chunked_src(2), dst_p, zeros128)
    xr = _mm_last(a4, dinv, b4)
    return xr[:N]


# speedup vs baseline: 7.7907x; 7.7907x over previous
import functools

import jax
import jax.numpy as jnp
from jax import lax
from jax.experimental import pallas as pl
from jax.experimental.pallas import tpu as pltpu
from jax.experimental.pallas import tpu_sc as plsc

N = 10000
IN_DIM = 256

NR = 10240
NC = 2
NS = 16
EB = 128
RPS = NR // NS
BR = 512


def _agg_body(C, F, NBt, table_C, split, table, pk, zeros_in, out,
              pk_v, isa, ida, isb, idb, buf_a, buf_b, zbuf, acc, sem_a, sem_b):
    cid = lax.axis_index("c")
    sid = lax.axis_index("s")
    rlo = sid * RPS
    cpc = C // NC
    nb = NBt // NC if split else NBt
    pltpu.sync_copy(pk.at[sid], pk_v)
    pltpu.sync_copy(zeros_in, zbuf)

    def unpack(j, off, si, di):
        for k in range(EB // 16):
            v = pk_v[j, pl.ds(k * 16, 16)]
            si[pl.ds(k * 16, 16)] = lax.shift_right_logical(v, 16) + off
            di[pl.ds(k * 16, 16)] = lax.bitwise_and(v, 0xFFFF)

    for local in range(cpc):
        chunk = cid * cpc + local
        off = chunk * NR if table_C == C else 0
        jbase = cid * nb if split else 0
        for z in range(RPS // 16):
            pltpu.sync_copy(zbuf, acc.at[pl.ds(rlo + z * 16, 16)])
        plsc.subcore_barrier()
        unpack(jbase, off, isa, ida)
        pltpu.async_copy(table.at[isa], buf_a, sem_a)
        unpack(jbase + 1, off, isb, idb)
        pltpu.async_copy(table.at[isb], buf_b, sem_b)

        def pair(p, carry):
            j0 = jbase + 2 * p
            pltpu.make_async_copy(table.at[isa], buf_a, sem_a).wait()
            pltpu.sync_copy(buf_a, acc.at[ida], add=True)
            unpack(j0 + 2, off, isa, ida)
            pltpu.async_copy(table.at[isa], buf_a, sem_a)
            pltpu.make_async_copy(table.at[isb], buf_b, sem_b).wait()
            pltpu.sync_copy(buf_b, acc.at[idb], add=True)
            unpack(j0 + 3, off, isb, idb)
            pltpu.async_copy(table.at[isb], buf_b, sem_b)
            return carry

        lax.fori_loop(0, nb // 2 - 1, pair, 0)
        pltpu.make_async_copy(table.at[isa], buf_a, sem_a).wait()
        pltpu.sync_copy(buf_a, acc.at[ida], add=True)
        pltpu.make_async_copy(table.at[isb], buf_b, sem_b).wait()
        pltpu.sync_copy(buf_b, acc.at[idb], add=True)
        plsc.subcore_barrier()
        pltpu.sync_copy(acc.at[pl.ds(rlo, RPS)],
                        out.at[chunk, pl.ds(rlo, RPS)])


@functools.lru_cache(maxsize=None)
def _make_agg(C, F, NBt, table_C, split):
    mesh = plsc.VectorSubcoreMesh(core_axis_name="c", subcore_axis_name="s")
    return pl.kernel(
        functools.partial(_agg_body, C, F, NBt, table_C, split),
        out_type=jax.ShapeDtypeStruct((C, NR, F), jnp.float32),
        mesh=mesh,
        scratch_types=[
            pltpu.VMEM((NBt, EB), jnp.int32),
            pltpu.VMEM((EB,), jnp.int32),
            pltpu.VMEM((EB,), jnp.int32),
            pltpu.VMEM((EB,), jnp.int32),
            pltpu.VMEM((EB,), jnp.int32),
            pltpu.VMEM((EB, F), jnp.float32),
            pltpu.VMEM((EB, F), jnp.float32),
            pltpu.VMEM((16, F), jnp.float32),
            pltpu.VMEM_SHARED((NR, F), jnp.float32),
            pltpu.SemaphoreType.DMA,
            pltpu.SemaphoreType.DMA,
        ],
    )


def _mm_first(x_pad, dinv, W, C_out, F_out):
    d_in = x_pad.shape[1]
    d_out = W.shape[1]

    def body(x_ref, dv_ref, w_ref, out_ref):
        res = jnp.dot(x_ref[...], w_ref[...],
                      preferred_element_type=jnp.float32) * dv_ref[...]
        for c2 in range(C_out):
            out_ref[c2] = res[:, c2 * F_out:(c2 + 1) * F_out]

    return pl.pallas_call(
        body,
        grid=(NR // BR,),
        in_specs=[
            pl.BlockSpec((BR, d_in), lambda i: (i, 0)),
            pl.BlockSpec((BR, 1), lambda i: (i, 0)),
            pl.BlockSpec((d_in, d_out), lambda i: (0, 0)),
        ],
        out_specs=pl.BlockSpec((C_out, BR, F_out), lambda i: (0, i, 0)),
        out_shape=jax.ShapeDtypeStruct((C_out, NR, F_out), jnp.float32),
    )(x_pad, dinv, W)


def _mm_mid(agg, dinv, b_prev, W, C_in, F_in, C_out, F_out, sum_in=False):
    d_out = W.shape[1]
    if sum_in:
        w_r = W.reshape(1, F_in, d_out)
        b_r = b_prev.reshape(1, 1, F_in)
    else:
        w_r = W.reshape(C_in, F_in, d_out)
        b_r = b_prev.reshape(C_in, 1, F_in)

    def body(a_ref, dv_ref, b_ref, w_ref, out_ref):
        dv = dv_ref[...]
        if sum_in:
            asum = a_ref[0]
            for c in range(1, C_in):
                asum = asum + a_ref[c]
            xc = jnp.maximum(asum * dv + b_ref[0], 0.0)
            acc = jnp.dot(xc, w_ref[0], preferred_element_type=jnp.float32)
        else:
            acc = jnp.zeros((BR, d_out), jnp.float32)
            for c in range(C_in):
                xc = jnp.maximum(a_ref[c] * dv + b_ref[c], 0.0)
                acc = acc + jnp.dot(xc, w_ref[c],
                                    preferred_element_type=jnp.float32)
        res = acc * dv
        for c2 in range(C_out):
            out_ref[c2] = res[:, c2 * F_out:(c2 + 1) * F_out]

    wc = 1 if sum_in else C_in
    return pl.pallas_call(
        body,
        grid=(NR // BR,),
        in_specs=[
            pl.BlockSpec((C_in, BR, F_in), lambda i: (0, i, 0)),
            pl.BlockSpec((BR, 1), lambda i: (i, 0)),
            pl.BlockSpec((wc, 1, F_in), lambda i: (0, 0, 0)),
            pl.BlockSpec((wc, F_in, d_out), lambda i: (0, 0, 0)),
        ],
        out_specs=pl.BlockSpec((C_out, BR, F_out), lambda i: (0, i, 0)),
        out_shape=jax.ShapeDtypeStruct((C_out, NR, F_out), jnp.float32),
    )(agg, dinv, b_r, w_r)


def _mm_last(agg, dinv, b4):
    b_r = b4.reshape(2, 1, 128)

    def body(a_ref, dv_ref, b_ref, out_ref):
        dv = dv_ref[...]
        for c in range(2):
            out_ref[:, c * 128:(c + 1) * 128] = a_ref[c] * dv + b_ref[c]

    return pl.pallas_call(
        body,
        grid=(NR // BR,),
        in_specs=[
            pl.BlockSpec((2, BR, 128), lambda i: (0, i, 0)),
            pl.BlockSpec((BR, 1), lambda i: (i, 0)),
            pl.BlockSpec((2, 1, 128), lambda i: (0, 0, 0)),
        ],
        out_specs=pl.BlockSpec((BR, 256), lambda i: (i, 0)),
        out_shape=jax.ShapeDtypeStruct((NR, 256), jnp.float32),
    )(agg, dinv, b_r)


def kernel(x, edge_index, W1, b1, W2, b2, W3, b3, W4, b4):
    E = edge_index.shape[1]
    loop = jnp.arange(N, dtype=jnp.int32)
    src_f = jnp.concatenate([edge_index[0], loop])
    dst_f = jnp.concatenate([edge_index[1], loop])
    e_full = E + N
    nbt = -(-e_full // (NS * EB))
    nbt = (nbt + 3) // 4 * 4
    pad = NS * nbt * EB - e_full
    packed = jnp.concatenate(
        [src_f * 65536 + dst_f,
         jnp.full((pad,), N * 65536 + N, jnp.int32)]).reshape(NS, nbt, EB)

    zeros128 = jnp.zeros((16, 128), jnp.float32)

    ones_t = jnp.ones((NR, 128), jnp.float32)
    degs = _make_agg(2, 128, nbt, 1, True)(ones_t, packed, zeros128)
    deg = degs[0, :, 0] + degs[1, :, 0]
    valid = (jnp.arange(NR) < N) & (deg > 0)
    dinv = jnp.where(valid, 1.0 / jnp.sqrt(jnp.maximum(deg, 1.0)), 0.0)
    dinv = dinv.reshape(NR, 1).astype(jnp.float32)

    x_pad = jnp.concatenate(
        [x, jnp.zeros((NR - N, IN_DIM), jnp.float32)], axis=0)

    t1 = _mm_first(x_pad, dinv, W1, 4, 128)
    a1 = _make_agg(4, 128, nbt, 4, False)(
        t1.reshape(4 * NR, 128), packed, zeros128)
    t2 = _mm_mid(a1, dinv, b1, W2, 4, 128, 1, 128)
    a2 = _make_agg(2, 128, nbt, 1, True)(
        t2.reshape(NR, 128), packed, zeros128)
    t3 = _mm_mid(a2, dinv, b2, W3, 2, 128, 4, 128, sum_in=True)
    a3 = _make_agg(4, 128, nbt, 4, False)(
        t3.reshape(4 * NR, 128), packed, zeros128)
    t4 = _mm_mid(a3, dinv, b3, W4, 4, 128, 2, 128)
    a4 = _make_agg(2, 128, nbt, 2, False)(
        t4.reshape(2 * NR, 128), packed, zeros128)
    xr = _mm_last(a4, dinv, b4)
    return xr[:N]


# speedup vs baseline: 11.2979x; 1.4502x over previous
import functools

import jax
import jax.numpy as jnp
from jax import lax
from jax.experimental import pallas as pl
from jax.experimental.pallas import tpu as pltpu
from jax.experimental.pallas import tpu_sc as plsc

N = 10000
IN_DIM = 256

NR = 10240
NC = 2
NS = 16
EB = 128
RPS = NR // NS
BR = 512


def _agg_body(C, F, NBt, table_C, split, table, pk, zeros_in, out,
              pk_v, isa, ida, isb, idb, buf_a, buf_b, zbuf, acc,
              sem_ga, sem_gb, sem_sa, sem_sb):
    cid = lax.axis_index("c")
    sid = lax.axis_index("s")
    rlo = sid * RPS
    cpc = C // NC
    nb = NBt // NC if split else NBt
    pltpu.sync_copy(pk.at[sid], pk_v)
    pltpu.sync_copy(zeros_in, zbuf)

    def unpack(j, off, si, di):
        for k in range(EB // 16):
            v = pk_v[j, pl.ds(k * 16, 16)]
            si[pl.ds(k * 16, 16)] = lax.shift_right_logical(v, 16) + off
            di[pl.ds(k * 16, 16)] = lax.bitwise_and(v, 0xFFFF)

    for local in range(cpc):
        chunk = cid * cpc + local
        off = chunk * NR if table_C == C else 0
        jbase = cid * nb if split else 0
        for z in range(RPS // 16):
            pltpu.sync_copy(zbuf, acc.at[pl.ds(rlo + z * 16, 16)])
        plsc.subcore_barrier()
        unpack(jbase, off, isa, ida)
        pltpu.async_copy(table.at[isa], buf_a, sem_ga)
        unpack(jbase + 1, off, isb, idb)
        pltpu.async_copy(table.at[isb], buf_b, sem_gb)

        def pair(p, carry):
            j0 = jbase + 2 * p
            pltpu.make_async_copy(table.at[isa], buf_a, sem_ga).wait()
            sca = pltpu.async_copy(buf_a, acc.at[ida], sem_sa, add=True)
            pltpu.make_async_copy(table.at[isb], buf_b, sem_gb).wait()
            scb = pltpu.async_copy(buf_b, acc.at[idb], sem_sb, add=True)
            sca.wait()
            unpack(j0 + 2, off, isa, ida)
            pltpu.async_copy(table.at[isa], buf_a, sem_ga)
            scb.wait()
            unpack(j0 + 3, off, isb, idb)
            pltpu.async_copy(table.at[isb], buf_b, sem_gb)
            return carry

        lax.fori_loop(0, nb // 2 - 1, pair, 0)
        pltpu.make_async_copy(table.at[isa], buf_a, sem_ga).wait()
        sca = pltpu.async_copy(buf_a, acc.at[ida], sem_sa, add=True)
        pltpu.make_async_copy(table.at[isb], buf_b, sem_gb).wait()
        scb = pltpu.async_copy(buf_b, acc.at[idb], sem_sb, add=True)
        sca.wait()
        scb.wait()
        plsc.subcore_barrier()
        pltpu.sync_copy(acc.at[pl.ds(rlo, RPS)],
                        out.at[chunk, pl.ds(rlo, RPS)])


@functools.lru_cache(maxsize=None)
def _make_agg(C, F, NBt, table_C, split):
    mesh = plsc.VectorSubcoreMesh(core_axis_name="c", subcore_axis_name="s")
    return pl.kernel(
        functools.partial(_agg_body, C, F, NBt, table_C, split),
        out_type=jax.ShapeDtypeStruct((C, NR, F), jnp.float32),
        mesh=mesh,
        scratch_types=[
            pltpu.VMEM((NBt, EB), jnp.int32),
            pltpu.VMEM((EB,), jnp.int32),
            pltpu.VMEM((EB,), jnp.int32),
            pltpu.VMEM((EB,), jnp.int32),
            pltpu.VMEM((EB,), jnp.int32),
            pltpu.VMEM((EB, F), jnp.float32),
            pltpu.VMEM((EB, F), jnp.float32),
            pltpu.VMEM((16, F), jnp.float32),
            pltpu.VMEM_SHARED((NR, F), jnp.float32),
            pltpu.SemaphoreType.DMA,
            pltpu.SemaphoreType.DMA,
            pltpu.SemaphoreType.DMA,
            pltpu.SemaphoreType.DMA,
        ],
    )


def _deg_body(NBt, pk, ones_in, zeros_in, out,
              pk_v, ida, idb, ones_v, zbuf, acc, sem_sa, sem_sb):
    cid = lax.axis_index("c")
    sid = lax.axis_index("s")
    rlo = sid * RPS
    nb = NBt // NC
    jbase = cid * nb
    pltpu.sync_copy(pk.at[sid], pk_v)
    pltpu.sync_copy(ones_in, ones_v)
    pltpu.sync_copy(zeros_in, zbuf)

    def unpack(j, di):
        for k in range(EB // 16):
            di[pl.ds(k * 16, 16)] = lax.bitwise_and(
                pk_v[j, pl.ds(k * 16, 16)], 0xFFFF)

    for z in range(RPS // 16):
        pltpu.sync_copy(zbuf, acc.at[pl.ds(rlo + z * 16, 16)])
    plsc.subcore_barrier()
    unpack(jbase, ida)
    pltpu.async_copy(ones_v, acc.at[ida], sem_sa, add=True)
    unpack(jbase + 1, idb)
    pltpu.async_copy(ones_v, acc.at[idb], sem_sb, add=True)

    def pair(p, carry):
        j0 = jbase + 2 * p
        pltpu.make_async_copy(ones_v, acc.at[ida], sem_sa).wait()
        unpack(j0 + 2, ida)
        pltpu.async_copy(ones_v, acc.at[ida], sem_sa, add=True)
        pltpu.make_async_copy(ones_v, acc.at[idb], sem_sb).wait()
        unpack(j0 + 3, idb)
        pltpu.async_copy(ones_v, acc.at[idb], sem_sb, add=True)
        return carry

    lax.fori_loop(0, nb // 2 - 1, pair, 0)
    pltpu.make_async_copy(ones_v, acc.at[ida], sem_sa).wait()
    pltpu.make_async_copy(ones_v, acc.at[idb], sem_sb).wait()
    plsc.subcore_barrier()
    pltpu.sync_copy(acc.at[pl.ds(rlo, RPS)], out.at[cid, pl.ds(rlo, RPS)])


@functools.lru_cache(maxsize=None)
def _make_deg(NBt):
    mesh = plsc.VectorSubcoreMesh(core_axis_name="c", subcore_axis_name="s")
    return pl.kernel(
        functools.partial(_deg_body, NBt),
        out_type=jax.ShapeDtypeStruct((NC, NR, 128), jnp.float32),
        mesh=mesh,
        scratch_types=[
            pltpu.VMEM((NBt, EB), jnp.int32),
            pltpu.VMEM((EB,), jnp.int32),
            pltpu.VMEM((EB,), jnp.int32),
            pltpu.VMEM((EB, 128), jnp.float32),
            pltpu.VMEM((16, 128), jnp.float32),
            pltpu.VMEM_SHARED((NR, 128), jnp.float32),
            pltpu.SemaphoreType.DMA,
            pltpu.SemaphoreType.DMA,
        ],
    )


def _mm_first(x_pad, dinv, W, C_out, F_out):
    d_in = x_pad.shape[1]
    d_out = W.shape[1]

    def body(x_ref, dv_ref, w_ref, out_ref):
        res = jnp.dot(x_ref[...], w_ref[...],
                      preferred_element_type=jnp.float32) * dv_ref[...]
        for c2 in range(C_out):
            out_ref[c2] = res[:, c2 * F_out:(c2 + 1) * F_out]

    return pl.pallas_call(
        body,
        grid=(NR // BR,),
        in_specs=[
            pl.BlockSpec((BR, d_in), lambda i: (i, 0)),
            pl.BlockSpec((BR, 1), lambda i: (i, 0)),
            pl.BlockSpec((d_in, d_out), lambda i: (0, 0)),
        ],
        out_specs=pl.BlockSpec((C_out, BR, F_out), lambda i: (0, i, 0)),
        out_shape=jax.ShapeDtypeStruct((C_out, NR, F_out), jnp.float32),
    )(x_pad, dinv, W)


def _mm_mid(agg, tbl, dinv, b_prev, W, C_in, F_in, C_out, F_out,
            sum_in=False):
    d_out = W.shape[1]
    tc = 1 if sum_in else C_in
    w_r = W.reshape(tc, F_in, d_out)
    b_r = b_prev.reshape(tc, 1, F_in)

    def body(a_ref, t_ref, dv_ref, b_ref, w_ref, out_ref):
        dv = dv_ref[...]
        if sum_in:
            asum = t_ref[0]
            for c in range(C_in):
                asum = asum + a_ref[c]
            xc = jnp.maximum(asum * dv + b_ref[0], 0.0)
            acc = jnp.dot(xc, w_ref[0], preferred_element_type=jnp.float32)
        else:
            acc = jnp.zeros((BR, d_out), jnp.float32)
            for c in range(C_in):
                xc = jnp.maximum((a_ref[c] + t_ref[c]) * dv + b_ref[c], 0.0)
                acc = acc + jnp.dot(xc, w_ref[c],
                                    preferred_element_type=jnp.float32)
        res = acc * dv
        for c2 in range(C_out):
            out_ref[c2] = res[:, c2 * F_out:(c2 + 1) * F_out]

    return pl.pallas_call(
        body,
        grid=(NR // BR,),
        in_specs=[
            pl.BlockSpec((C_in, BR, F_in), lambda i: (0, i, 0)),
            pl.BlockSpec((tc, BR, F_in), lambda i: (0, i, 0)),
            pl.BlockSpec((BR, 1), lambda i: (i, 0)),
            pl.BlockSpec((tc, 1, F_in), lambda i: (0, 0, 0)),
            pl.BlockSpec((tc, F_in, d_out), lambda i: (0, 0, 0)),
        ],
        out_specs=pl.BlockSpec((C_out, BR, F_out), lambda i: (0, i, 0)),
        out_shape=jax.ShapeDtypeStruct((C_out, NR, F_out), jnp.float32),
    )(agg, tbl, dinv, b_r, w_r)


def _mm_last(agg, tbl, dinv, b4):
    b_r = b4.reshape(2, 1, 128)

    def body(a_ref, t_ref, dv_ref, b_ref, out_ref):
        dv = dv_ref[...]
        for c in range(2):
            out_ref[:, c * 128:(c + 1) * 128] = \
                (a_ref[c] + t_ref[c]) * dv + b_ref[c]

    return pl.pallas_call(
        body,
        grid=(NR // BR,),
        in_specs=[
            pl.BlockSpec((2, BR, 128), lambda i: (0, i, 0)),
            pl.BlockSpec((2, BR, 128), lambda i: (0, i, 0)),
            pl.BlockSpec((BR, 1), lambda i: (i, 0)),
            pl.BlockSpec((2, 1, 128), lambda i: (0, 0, 0)),
        ],
        out_specs=pl.BlockSpec((BR, 256), lambda i: (i, 0)),
        out_shape=jax.ShapeDtypeStruct((NR, 256), jnp.float32),
    )(agg, tbl, dinv, b_r)


def kernel(x, edge_index, W1, b1, W2, b2, W3, b3, W4, b4):
    E = edge_index.shape[1]
    src_f = edge_index[0]
    dst_f = edge_index[1]
    nbt = -(-E // (NS * EB))
    nbt = (nbt + 3) // 4 * 4
    pad = NS * nbt * EB - E
    dummy = N + jnp.arange(pad, dtype=jnp.int32) % (NR - N)
    flat = jnp.concatenate(
        [src_f * 65536 + dst_f, dummy * 65536 + dummy])
    packed = flat.reshape(nbt * EB, NS).T.reshape(NS, nbt, EB)

    zeros128 = jnp.zeros((16, 128), jnp.float32)
    ones128 = jnp.ones((EB, 128), jnp.float32)

    degs = _make_deg(nbt)(packed, ones128, zeros128)
    deg = degs[0, :, 0] + degs[1, :, 0] + 1.0
    valid = jnp.arange(NR) < N
    dinv = jnp.where(valid, 1.0 / jnp.sqrt(deg), 0.0)
    dinv = dinv.reshape(NR, 1).astype(jnp.float32)

    x_pad = jnp.concatenate(
        [x, jnp.zeros((NR - N, IN_DIM), jnp.float32)], axis=0)

    t1 = _mm_first(x_pad, dinv, W1, 4, 128)
    a1 = _make_agg(4, 128, nbt, 4, False)(
        t1.reshape(4 * NR, 128), packed, zeros128)
    t2 = _mm_mid(a1, t1, dinv, b1, W2, 4, 128, 1, 128)
    a2 = _make_agg(2, 128, nbt, 1, True)(
        t2.reshape(NR, 128), packed, zeros128)
    t3 = _mm_mid(a2, t2, dinv, b2, W3, 2, 128, 4, 128, sum_in=True)
    a3 = _make_agg(4, 128, nbt, 4, False)(
        t3.reshape(4 * NR, 128), packed, zeros128)
    t4 = _mm_mid(a3, t3, dinv, b3, W4, 4, 128, 2, 128)
    a4 = _make_agg(2, 128, nbt, 2, False)(
        t4.reshape(2 * NR, 128), packed, zeros128)
    xr = _mm_last(a4, t4, dinv, b4)
    return xr[:N]


# speedup vs baseline: 13.8349x; 1.2246x over previous
import functools

import jax
import jax.numpy as jnp
from jax import lax
from jax.experimental import pallas as pl
from jax.experimental.pallas import tpu as pltpu
from jax.experimental.pallas import tpu_sc as plsc

N = 10000
IN_DIM = 256

NR = 10240
NC = 2
NS = 16
EB = 128
EBH = 64
RPS = NR // NS
BR = 512


def _agg_body(C, F, NBt, table_C, split, table, pk, zeros_in, out,
              pk_v, is0, id0, is1, id1, is2, id2, is3, id3,
              buf0, buf1, buf2, buf3, zbuf, acc,
              sg0, sg1, sg2, sg3, ss0, ss1, ss2, ss3):
    cid = lax.axis_index("c")
    sid = lax.axis_index("s")
    rlo = sid * RPS
    cpc = C // NC
    nbh = 2 * NBt
    nb = nbh // NC if split else nbh
    iss = [is0, is1, is2, is3]
    ids = [id0, id1, id2, id3]
    bufs = [buf0, buf1, buf2, buf3]
    sgs = [sg0, sg1, sg2, sg3]
    sss = [ss0, ss1, ss2, ss3]
    pltpu.sync_copy(pk.at[sid], pk_v)
    pltpu.sync_copy(zeros_in, zbuf)

    def unpack(j, off, si, di):
        for k in range(EBH // 16):
            v = pk_v[j // 2, pl.ds((j % 2) * EBH + k * 16, 16)]
            si[pl.ds(k * 16, 16)] = lax.shift_right_logical(v, 16) + off
            di[pl.ds(k * 16, 16)] = lax.bitwise_and(v, 0xFFFF)

    for local in range(cpc):
        chunk = cid * cpc + local
        off = chunk * NR if table_C == C else 0
        jbase = cid * nb if split else 0
        for z in range(RPS // 16):
            pltpu.async_copy(zbuf, acc.at[pl.ds(rlo + z * 16, 16)], sg0)
        for z in range(RPS // 16):
            pltpu.make_async_copy(
                zbuf, acc.at[pl.ds(rlo + z * 16, 16)], sg0).wait()
        plsc.subcore_barrier()
        for t in range(4):
            unpack(jbase + t, off, iss[t], ids[t])
            pltpu.async_copy(table.at[iss[t]], bufs[t], sgs[t])

        def grp(p, carry):
            j0 = jbase + 4 * p
            scs = []
            for t in range(4):
                pltpu.make_async_copy(table.at[iss[t]], bufs[t], sgs[t]).wait()
                scs.append(pltpu.async_copy(
                    bufs[t], acc.at[ids[t]], sss[t], add=True))
            for t in range(4):
                scs[t].wait()
                unpack(j0 + 4 + t, off, iss[t], ids[t])
                pltpu.async_copy(table.at[iss[t]], bufs[t], sgs[t])
            return carry

        lax.fori_loop(0, nb // 4 - 1, grp, 0)
        scs = []
        for t in range(4):
            pltpu.make_async_copy(table.at[iss[t]], bufs[t], sgs[t]).wait()
            scs.append(pltpu.async_copy(
                bufs[t], acc.at[ids[t]], sss[t], add=True))
        for t in range(4):
            scs[t].wait()
        plsc.subcore_barrier()
        pltpu.sync_copy(acc.at[pl.ds(rlo, RPS)],
                        out.at[chunk, pl.ds(rlo, RPS)])


@functools.lru_cache(maxsize=None)
def _make_agg(C, F, NBt, table_C, split):
    mesh = plsc.VectorSubcoreMesh(core_axis_name="c", subcore_axis_name="s")
    return pl.kernel(
        functools.partial(_agg_body, C, F, NBt, table_C, split),
        out_type=jax.ShapeDtypeStruct((C, NR, F), jnp.float32),
        mesh=mesh,
        scratch_types=[
            pltpu.VMEM((NBt, EB), jnp.int32)] +
        [pltpu.VMEM((EBH,), jnp.int32) for _ in range(8)] +
        [pltpu.VMEM((EBH, F), jnp.float32) for _ in range(4)] +
        [
            pltpu.VMEM((16, F), jnp.float32),
            pltpu.VMEM_SHARED((NR, F), jnp.float32),
        ] + [pltpu.SemaphoreType.DMA] * 8,
    )


def _deg_body(NBt, pk, ones_in, zeros_in, out,
              pk_v, ida, idb, ones_v, zbuf, acc, sem_sa, sem_sb):
    cid = lax.axis_index("c")
    sid = lax.axis_index("s")
    rlo = sid * RPS
    nb = 2 * NBt // NC
    jbase = cid * nb
    pltpu.sync_copy(pk.at[sid], pk_v)
    pltpu.sync_copy(ones_in, ones_v)
    pltpu.sync_copy(zeros_in, zbuf)

    def unpack(j, di):
        for k in range(EBH // 16):
            di[pl.ds(k * 16, 16)] = lax.bitwise_and(
                pk_v[j // 2, pl.ds((j % 2) * EBH + k * 16, 16)], 0xFFFF)

    for z in range(RPS // 16):
        pltpu.async_copy(zbuf, acc.at[pl.ds(rlo + z * 16, 16)], sem_sa)
    for z in range(RPS // 16):
        pltpu.make_async_copy(
            zbuf, acc.at[pl.ds(rlo + z * 16, 16)], sem_sa).wait()
    plsc.subcore_barrier()
    unpack(jbase, ida)
    pltpu.async_copy(ones_v, acc.at[ida], sem_sa, add=True)
    unpack(jbase + 1, idb)
    pltpu.async_copy(ones_v, acc.at[idb], sem_sb, add=True)

    def pair(p, carry):
        j0 = jbase + 2 * p
        pltpu.make_async_copy(ones_v, acc.at[ida], sem_sa).wait()
        unpack(j0 + 2, ida)
        pltpu.async_copy(ones_v, acc.at[ida], sem_sa, add=True)
        pltpu.make_async_copy(ones_v, acc.at[idb], sem_sb).wait()
        unpack(j0 + 3, idb)
        pltpu.async_copy(ones_v, acc.at[idb], sem_sb, add=True)
        return carry

    lax.fori_loop(0, nb // 2 - 1, pair, 0)
    pltpu.make_async_copy(ones_v, acc.at[ida], sem_sa).wait()
    pltpu.make_async_copy(ones_v, acc.at[idb], sem_sb).wait()
    plsc.subcore_barrier()
    pltpu.sync_copy(acc.at[pl.ds(rlo, RPS)], out.at[cid, pl.ds(rlo, RPS)])


@functools.lru_cache(maxsize=None)
def _make_deg(NBt):
    mesh = plsc.VectorSubcoreMesh(core_axis_name="c", subcore_axis_name="s")
    return pl.kernel(
        functools.partial(_deg_body, NBt),
        out_type=jax.ShapeDtypeStruct((NC, NR, 128), jnp.float32),
        mesh=mesh,
        scratch_types=[
            pltpu.VMEM((NBt, EB), jnp.int32),
            pltpu.VMEM((EBH,), jnp.int32),
            pltpu.VMEM((EBH,), jnp.int32),
            pltpu.VMEM((EBH, 128), jnp.float32),
            pltpu.VMEM((16, 128), jnp.float32),
            pltpu.VMEM_SHARED((NR, 128), jnp.float32),
            pltpu.SemaphoreType.DMA,
            pltpu.SemaphoreType.DMA,
        ],
    )


def _mm_first(x_pad, dinv, W, C_out, F_out):
    d_in = x_pad.shape[1]
    d_out = W.shape[1]

    def body(x_ref, dv_ref, w_ref, out_ref):
        res = jnp.dot(x_ref[...], w_ref[...],
                      preferred_element_type=jnp.float32) * dv_ref[...]
        for c2 in range(C_out):
            out_ref[c2] = res[:, c2 * F_out:(c2 + 1) * F_out]

    return pl.pallas_call(
        body,
        grid=(NR // BR,),
        in_specs=[
            pl.BlockSpec((BR, d_in), lambda i: (i, 0)),
            pl.BlockSpec((BR, 1), lambda i: (i, 0)),
            pl.BlockSpec((d_in, d_out), lambda i: (0, 0)),
        ],
        out_specs=pl.BlockSpec((C_out, BR, F_out), lambda i: (0, i, 0)),
        out_shape=jax.ShapeDtypeStruct((C_out, NR, F_out), jnp.float32),
    )(x_pad, dinv, W)


def _mm_mid(agg, tbl, dinv, b_prev, W, C_in, F_in, C_out, F_out,
            sum_in=False):
    d_out = W.shape[1]
    tc = 1 if sum_in else C_in
    w_r = W.reshape(tc, F_in, d_out)
    b_r = b_prev.reshape(tc, 1, F_in)

    def body(a_ref, t_ref, dv_ref, b_ref, w_ref, out_ref):
        dv = dv_ref[...]
        if sum_in:
            asum = t_ref[0]
            for c in range(C_in):
                asum = asum + a_ref[c]
            xc = jnp.maximum(asum * dv + b_ref[0], 0.0)
            acc = jnp.dot(xc, w_ref[0], preferred_element_type=jnp.float32)
        else:
            acc = jnp.zeros((BR, d_out), jnp.float32)
            for c in range(C_in):
                xc = jnp.maximum((a_ref[c] + t_ref[c]) * dv + b_ref[c], 0.0)
                acc = acc + jnp.dot(xc, w_ref[c],
                                    preferred_element_type=jnp.float32)
        res = acc * dv
        for c2 in range(C_out):
            out_ref[c2] = res[:, c2 * F_out:(c2 + 1) * F_out]

    return pl.pallas_call(
        body,
        grid=(NR // BR,),
        in_specs=[
            pl.BlockSpec((C_in, BR, F_in), lambda i: (0, i, 0)),
            pl.BlockSpec((tc, BR, F_in), lambda i: (0, i, 0)),
            pl.BlockSpec((BR, 1), lambda i: (i, 0)),
            pl.BlockSpec((tc, 1, F_in), lambda i: (0, 0, 0)),
            pl.BlockSpec((tc, F_in, d_out), lambda i: (0, 0, 0)),
        ],
        out_specs=pl.BlockSpec((C_out, BR, F_out), lambda i: (0, i, 0)),
        out_shape=jax.ShapeDtypeStruct((C_out, NR, F_out), jnp.float32),
    )(agg, tbl, dinv, b_r, w_r)


def _mm_last(agg, tbl, dinv, b4):
    b_r = b4.reshape(2, 1, 128)

    def body(a_ref, t_ref, dv_ref, b_ref, out_ref):
        dv = dv_ref[...]
        for c in range(2):
            out_ref[:, c * 128:(c + 1) * 128] = \
                (a_ref[c] + t_ref[c]) * dv + b_ref[c]

    return pl.pallas_call(
        body,
        grid=(NR // BR,),
        in_specs=[
            pl.BlockSpec((2, BR, 128), lambda i: (0, i, 0)),
            pl.BlockSpec((2, BR, 128), lambda i: (0, i, 0)),
            pl.BlockSpec((BR, 1), lambda i: (i, 0)),
            pl.BlockSpec((2, 1, 128), lambda i: (0, 0, 0)),
        ],
        out_specs=pl.BlockSpec((BR, 256), lambda i: (i, 0)),
        out_shape=jax.ShapeDtypeStruct((NR, 256), jnp.float32),
    )(agg, tbl, dinv, b_r)


def kernel(x, edge_index, W1, b1, W2, b2, W3, b3, W4, b4):
    E = edge_index.shape[1]
    src_f = edge_index[0]
    dst_f = edge_index[1]
    nbt = -(-E // (NS * EB))
    nbt = (nbt + 3) // 4 * 4
    pad = NS * nbt * EB - E
    dummy = N + jnp.arange(pad, dtype=jnp.int32) % (NR - N)
    flat = jnp.concatenate(
        [src_f * 65536 + dst_f, dummy * 65536 + dummy])
    packed = flat.reshape(nbt * EB, NS).T.reshape(NS, nbt, EB)

    zeros128 = jnp.zeros((16, 128), jnp.float32)
    ones128 = jnp.ones((EBH, 128), jnp.float32)

    degs = _make_deg(nbt)(packed, ones128, zeros128)
    deg = degs[0, :, 0] + degs[1, :, 0] + 1.0
    valid = jnp.arange(NR) < N
    dinv = jnp.where(valid, 1.0 / jnp.sqrt(deg), 0.0)
    dinv = dinv.reshape(NR, 1).astype(jnp.float32)

    x_pad = jnp.concatenate(
        [x, jnp.zeros((NR - N, IN_DIM), jnp.float32)], axis=0)

    t1 = _mm_first(x_pad, dinv, W1, 4, 128)
    a1 = _make_agg(4, 128, nbt, 4, False)(
        t1.reshape(4 * NR, 128), packed, zeros128)
    t2 = _mm_mid(a1, t1, dinv, b1, W2, 4, 128, 1, 128)
    a2 = _make_agg(2, 128, nbt, 1, True)(
        t2.reshape(NR, 128), packed, zeros128)
    t3 = _mm_mid(a2, t2, dinv, b2, W3, 2, 128, 4, 128, sum_in=True)
    a3 = _make_agg(4, 128, nbt, 4, False)(
        t3.reshape(4 * NR, 128), packed, zeros128)
    t4 = _mm_mid(a3, t3, dinv, b3, W4, 4, 128, 2, 128)
    a4 = _make_agg(2, 128, nbt, 2, False)(
        t4.reshape(2 * NR, 128), packed, zeros128)
    xr = _mm_last(a4, t4, dinv, b4)
    return xr[:N]


# speedup vs baseline: 13.9601x; 1.0091x over previous
import functools

import jax
import jax.numpy as jnp
from jax import lax
from jax.experimental import pallas as pl
from jax.experimental.pallas import tpu as pltpu
from jax.experimental.pallas import tpu_sc as plsc

N = 10000
IN_DIM = 256

NR = 10240
NC = 2
NS = 16
EB = 128
EBH = 64
RPS = NR // NS
BR = 512


def _agg_body(C, F, NBt, table_C, split, table, pk, zeros_in, out,
              pk_v, is0, id0, is1, id1, is2, id2, is3, id3,
              buf0, buf1, buf2, buf3, zbuf, acc,
              sg0, sg1, sg2, sg3, ss0, ss1, ss2, ss3):
    cid = lax.axis_index("c")
    sid = lax.axis_index("s")
    rlo = sid * RPS
    cpc = C // NC
    nbh = 2 * NBt
    nb = nbh // NC if split else nbh
    iss = [is0, is1, is2, is3]
    ids = [id0, id1, id2, id3]
    bufs = [buf0, buf1, buf2, buf3]
    sgs = [sg0, sg1, sg2, sg3]
    sss = [ss0, ss1, ss2, ss3]
    pltpu.sync_copy(pk.at[sid], pk_v)
    pltpu.sync_copy(zeros_in, zbuf)

    def unpack(j, off, si, di):
        for k in range(EBH // 16):
            v = pk_v[j // 2, pl.ds((j % 2) * EBH + k * 16, 16)]
            si[pl.ds(k * 16, 16)] = lax.shift_right_logical(v, 16) + off
            di[pl.ds(k * 16, 16)] = lax.bitwise_and(v, 0xFFFF)

    for local in range(cpc):
        chunk = cid * cpc + local
        off = chunk * NR if table_C == C else 0
        jbase = cid * nb if split else 0
        for z in range(RPS // 16):
            pltpu.async_copy(zbuf, acc.at[pl.ds(rlo + z * 16, 16)], sg0)
        for z in range(RPS // 16):
            pltpu.make_async_copy(
                zbuf, acc.at[pl.ds(rlo + z * 16, 16)], sg0).wait()
        plsc.subcore_barrier()
        for t in range(4):
            unpack(jbase + t, off, iss[t], ids[t])
            pltpu.async_copy(table.at[iss[t]], bufs[t], sgs[t])

        def grp(p, carry):
            j0 = jbase + 4 * p
            scs = []
            for t in range(4):
                pltpu.make_async_copy(table.at[iss[t]], bufs[t], sgs[t]).wait()
                scs.append(pltpu.async_copy(
                    bufs[t], acc.at[ids[t]], sss[t], add=True))
            for t in range(4):
                scs[t].wait()
                unpack(j0 + 4 + t, off, iss[t], ids[t])
                pltpu.async_copy(table.at[iss[t]], bufs[t], sgs[t])
            return carry

        lax.fori_loop(0, nb // 4 - 1, grp, 0)
        scs = []
        for t in range(4):
            pltpu.make_async_copy(table.at[iss[t]], bufs[t], sgs[t]).wait()
            scs.append(pltpu.async_copy(
                bufs[t], acc.at[ids[t]], sss[t], add=True))
        for t in range(4):
            scs[t].wait()
        plsc.subcore_barrier()
        pltpu.sync_copy(acc.at[pl.ds(rlo, RPS)],
                        out.at[chunk, pl.ds(rlo, RPS)])


@functools.lru_cache(maxsize=None)
def _make_agg(C, F, NBt, table_C, split):
    mesh = plsc.VectorSubcoreMesh(core_axis_name="c", subcore_axis_name="s")
    return pl.kernel(
        functools.partial(_agg_body, C, F, NBt, table_C, split),
        out_type=jax.ShapeDtypeStruct((C, NR, F), jnp.float32),
        mesh=mesh,
        scratch_types=[
            pltpu.VMEM((NBt, EB), jnp.int32)] +
        [pltpu.VMEM((EBH,), jnp.int32) for _ in range(8)] +
        [pltpu.VMEM((EBH, F), jnp.float32) for _ in range(4)] +
        [
            pltpu.VMEM((16, F), jnp.float32),
            pltpu.VMEM_SHARED((NR, F), jnp.float32),
        ] + [pltpu.SemaphoreType.DMA] * 8,
    )


def _deg_body(NBt, pk, ones_in, zeros_in, out,
              pk_v, ida, idb, ones_v, zbuf, acc, sem_sa, sem_sb):
    cid = lax.axis_index("c")
    sid = lax.axis_index("s")
    rlo = sid * RPS
    nb = 2 * NBt // NC
    jbase = cid * nb
    pltpu.sync_copy(pk.at[sid], pk_v)
    pltpu.sync_copy(ones_in, ones_v)
    pltpu.sync_copy(zeros_in, zbuf)

    def unpack(j, di):
        for k in range(EBH // 16):
            di[pl.ds(k * 16, 16)] = lax.bitwise_and(
                pk_v[j // 2, pl.ds((j % 2) * EBH + k * 16, 16)], 0xFFFF)

    for z in range(RPS // 16):
        pltpu.async_copy(zbuf, acc.at[pl.ds(rlo + z * 16, 16)], sem_sa)
    for z in range(RPS // 16):
        pltpu.make_async_copy(
            zbuf, acc.at[pl.ds(rlo + z * 16, 16)], sem_sa).wait()
    plsc.subcore_barrier()
    unpack(jbase, ida)
    pltpu.async_copy(ones_v, acc.at[ida], sem_sa, add=True)
    unpack(jbase + 1, idb)
    pltpu.async_copy(ones_v, acc.at[idb], sem_sb, add=True)

    def pair(p, carry):
        j0 = jbase + 2 * p
        pltpu.make_async_copy(ones_v, acc.at[ida], sem_sa).wait()
        unpack(j0 + 2, ida)
        pltpu.async_copy(ones_v, acc.at[ida], sem_sa, add=True)
        pltpu.make_async_copy(ones_v, acc.at[idb], sem_sb).wait()
        unpack(j0 + 3, idb)
        pltpu.async_copy(ones_v, acc.at[idb], sem_sb, add=True)
        return carry

    lax.fori_loop(0, nb // 2 - 1, pair, 0)
    pltpu.make_async_copy(ones_v, acc.at[ida], sem_sa).wait()
    pltpu.make_async_copy(ones_v, acc.at[idb], sem_sb).wait()
    plsc.subcore_barrier()
    pltpu.sync_copy(acc.at[pl.ds(rlo, RPS)], out.at[cid, pl.ds(rlo, RPS)])


@functools.lru_cache(maxsize=None)
def _make_deg(NBt):
    mesh = plsc.VectorSubcoreMesh(core_axis_name="c", subcore_axis_name="s")
    return pl.kernel(
        functools.partial(_deg_body, NBt),
        out_type=jax.ShapeDtypeStruct((NC, NR, 128), jnp.float32),
        mesh=mesh,
        scratch_types=[
            pltpu.VMEM((NBt, EB), jnp.int32),
            pltpu.VMEM((EBH,), jnp.int32),
            pltpu.VMEM((EBH,), jnp.int32),
            pltpu.VMEM((EBH, 128), jnp.float32),
            pltpu.VMEM((16, 128), jnp.float32),
            pltpu.VMEM_SHARED((NR, 128), jnp.float32),
            pltpu.SemaphoreType.DMA,
            pltpu.SemaphoreType.DMA,
        ],
    )


def _mm_u1(x_pad, W, C_out, F_out):
    d_in = x_pad.shape[1]

    def body(x_ref, w_ref, out_ref):
        res = jnp.dot(x_ref[...], w_ref[...],
                      preferred_element_type=jnp.float32)
        for c2 in range(C_out):
            out_ref[c2] = res[:, c2 * F_out:(c2 + 1) * F_out]

    return pl.pallas_call(
        body,
        grid=(NR // BR,),
        in_specs=[
            pl.BlockSpec((BR, d_in), lambda i: (i, 0)),
            pl.BlockSpec((d_in, C_out * F_out), lambda i: (0, 0)),
        ],
        out_specs=pl.BlockSpec((C_out, BR, F_out), lambda i: (0, i, 0)),
        out_shape=jax.ShapeDtypeStruct((C_out, NR, F_out), jnp.float32),
    )(x_pad, W)


def _scale_first(u1, degs, C):
    def body(u_ref, d_ref, t_ref, dv_ref):
        deg = d_ref[0, :, 0:1] + d_ref[1, :, 0:1] + 1.0
        dv = 1.0 / jnp.sqrt(deg)
        for c in range(C):
            t_ref[c] = u_ref[c] * dv
        dv_ref[...] = dv

    return pl.pallas_call(
        body,
        grid=(NR // BR,),
        in_specs=[
            pl.BlockSpec((C, BR, 128), lambda i: (0, i, 0)),
            pl.BlockSpec((2, BR, 128), lambda i: (0, i, 0)),
        ],
        out_specs=[
            pl.BlockSpec((C, BR, 128), lambda i: (0, i, 0)),
            pl.BlockSpec((BR, 1), lambda i: (i, 0)),
        ],
        out_shape=[
            jax.ShapeDtypeStruct((C, NR, 128), jnp.float32),
            jax.ShapeDtypeStruct((NR, 1), jnp.float32),
        ],
    )(u1, degs)


def _mm_mid(agg, tbl, dinv, b_prev, W, C_in, F_in, C_out, F_out,
            sum_in=False):
    d_out = W.shape[1]
    tc = 1 if sum_in else C_in
    w_r = W.reshape(tc, F_in, d_out)
    b_r = b_prev.reshape(tc, 1, F_in)

    def body(a_ref, t_ref, dv_ref, b_ref, w_ref, out_ref):
        dv = dv_ref[...]
        if sum_in:
            asum = t_ref[0]
            for c in range(C_in):
                asum = asum + a_ref[c]
            xc = jnp.maximum(asum * dv + b_ref[0], 0.0)
            acc = jnp.dot(xc, w_ref[0], preferred_element_type=jnp.float32)
        else:
            acc = jnp.zeros((BR, d_out), jnp.float32)
            for c in range(C_in):
                xc = jnp.maximum((a_ref[c] + t_ref[c]) * dv + b_ref[c], 0.0)
                acc = acc + jnp.dot(xc, w_ref[c],
                                    preferred_element_type=jnp.float32)
        res = acc * dv
        for c2 in range(C_out):
            out_ref[c2] = res[:, c2 * F_out:(c2 + 1) * F_out]

    return pl.pallas_call(
        body,
        grid=(NR // BR,),
        in_specs=[
            pl.BlockSpec((C_in, BR, F_in), lambda i: (0, i, 0)),
            pl.BlockSpec((tc, BR, F_in), lambda i: (0, i, 0)),
            pl.BlockSpec((BR, 1), lambda i: (i, 0)),
            pl.BlockSpec((tc, 1, F_in), lambda i: (0, 0, 0)),
            pl.BlockSpec((tc, F_in, d_out), lambda i: (0, 0, 0)),
        ],
        out_specs=pl.BlockSpec((C_out, BR, F_out), lambda i: (0, i, 0)),
        out_shape=jax.ShapeDtypeStruct((C_out, NR, F_out), jnp.float32),
    )(agg, tbl, dinv, b_r, w_r)


def _mm_last(agg, tbl, dinv, b4):
    b_r = b4.reshape(2, 1, 128)
    blk = 400

    def body(a_ref, t_ref, dv_ref, b_ref, out_ref):
        dv = dv_ref[...]
        for c in range(2):
            out_ref[:, c * 128:(c + 1) * 128] = \
                (a_ref[c] + t_ref[c]) * dv + b_ref[c]

    return pl.pallas_call(
        body,
        grid=(N // blk,),
        in_specs=[
            pl.BlockSpec((2, blk, 128), lambda i: (0, i, 0)),
            pl.BlockSpec((2, blk, 128), lambda i: (0, i, 0)),
            pl.BlockSpec((blk, 1), lambda i: (i, 0)),
            pl.BlockSpec((2, 1, 128), lambda i: (0, 0, 0)),
        ],
        out_specs=pl.BlockSpec((blk, 256), lambda i: (i, 0)),
        out_shape=jax.ShapeDtypeStruct((N, 256), jnp.float32),
    )(agg, tbl, dinv, b_r)


def kernel(x, edge_index, W1, b1, W2, b2, W3, b3, W4, b4):
    E = edge_index.shape[1]
    src_f = edge_index[0]
    dst_f = edge_index[1]
    nbt = -(-E // (NS * EB))
    nbt = (nbt + 3) // 4 * 4
    pad = NS * nbt * EB - E
    dummy = N + jnp.arange(pad, dtype=jnp.int32) % (NR - N)
    flat = jnp.concatenate(
        [src_f * 65536 + dst_f, dummy * 65536 + dummy])
    packed = flat.reshape(nbt * EB, NS).T.reshape(NS, nbt, EB)

    zeros128 = jnp.zeros((16, 128), jnp.float32)
    ones128 = jnp.ones((EBH, 128), jnp.float32)

    x_pad = jnp.concatenate(
        [x, jnp.zeros((NR - N, IN_DIM), jnp.float32)], axis=0)

    degs = _make_deg(nbt)(packed, ones128, zeros128)
    u1 = _mm_u1(x_pad, W1, 4, 128)
    t1, dinv = _scale_first(u1, degs, 4)
    a1 = _make_agg(4, 128, nbt, 4, False)(
        t1.reshape(4 * NR, 128), packed, zeros128)
    t2 = _mm_mid(a1, t1, dinv, b1, W2, 4, 128, 1, 128)
    a2 = _make_agg(2, 128, nbt, 1, True)(
        t2.reshape(NR, 128), packed, zeros128)
    t3 = _mm_mid(a2, t2, dinv, b2, W3, 2, 128, 4, 128, sum_in=True)
    a3 = _make_agg(4, 128, nbt, 4, False)(
        t3.reshape(4 * NR, 128), packed, zeros128)
    t4 = _mm_mid(a3, t3, dinv, b3, W4, 4, 128, 2, 128)
    a4 = _make_agg(2, 128, nbt, 2, False)(
        t4.reshape(2 * NR, 128), packed, zeros128)
    return _mm_last(a4, t4, dinv, b4)


# speedup vs baseline: 14.1897x; 1.0164x over previous
import functools

import jax
import jax.numpy as jnp
from jax import lax
from jax.experimental import pallas as pl
from jax.experimental.pallas import tpu as pltpu
from jax.experimental.pallas import tpu_sc as plsc

N = 10000
IN_DIM = 256

NR = 10240
NC = 2
NS = 16
EB = 128
EBH = 64
RPS = NR // NS
DW = 128
BR = 512


def _agg_body(C, F, NBt, table_C, split, table, pk, zeros_in, out,
              pk_v, is0, id0, is1, id1, is2, id2, is3, id3,
              buf0, buf1, buf2, buf3, zbuf, acc,
              sg0, sg1, sg2, sg3, ss0, ss1, ss2, ss3):
    cid = lax.axis_index("c")
    sid = lax.axis_index("s")
    rlo = sid * RPS
    cpc = C // NC
    nbh = 2 * NBt
    nb = nbh // NC if split else nbh
    iss = [is0, is1, is2, is3]
    ids = [id0, id1, id2, id3]
    bufs = [buf0, buf1, buf2, buf3]
    sgs = [sg0, sg1, sg2, sg3]
    sss = [ss0, ss1, ss2, ss3]
    pltpu.sync_copy(pk.at[sid], pk_v)
    pltpu.sync_copy(zeros_in, zbuf)

    def unpack(j, off, si, di):
        for k in range(EBH // 16):
            v = pk_v[j // 2, pl.ds((j % 2) * EBH + k * 16, 16)]
            si[pl.ds(k * 16, 16)] = lax.shift_right_logical(v, 16) + off
            di[pl.ds(k * 16, 16)] = lax.bitwise_and(v, 0xFFFF)

    for local in range(cpc):
        chunk = cid * cpc + local
        off = chunk * NR if table_C == C else 0
        jbase = cid * nb if split else 0
        for z in range(RPS // 16):
            pltpu.async_copy(zbuf, acc.at[pl.ds(rlo + z * 16, 16)], sg0)
        for z in range(RPS // 16):
            pltpu.make_async_copy(
                zbuf, acc.at[pl.ds(rlo + z * 16, 16)], sg0).wait()
        plsc.subcore_barrier()
        for t in range(4):
            unpack(jbase + t, off, iss[t], ids[t])
            pltpu.async_copy(table.at[iss[t]], bufs[t], sgs[t])

        def grp(p, carry):
            j0 = jbase + 4 * p
            scs = []
            for t in range(4):
                pltpu.make_async_copy(table.at[iss[t]], bufs[t], sgs[t]).wait()
                scs.append(pltpu.async_copy(
                    bufs[t], acc.at[ids[t]], sss[t], add=True))
            for t in range(4):
                scs[t].wait()
                unpack(j0 + 4 + t, off, iss[t], ids[t])
                pltpu.async_copy(table.at[iss[t]], bufs[t], sgs[t])
            return carry

        lax.fori_loop(0, nb // 4 - 1, grp, 0)
        scs = []
        for t in range(4):
            pltpu.make_async_copy(table.at[iss[t]], bufs[t], sgs[t]).wait()
            scs.append(pltpu.async_copy(
                bufs[t], acc.at[ids[t]], sss[t], add=True))
        for t in range(4):
            scs[t].wait()
        plsc.subcore_barrier()
        pltpu.sync_copy(acc.at[pl.ds(rlo, RPS)],
                        out.at[chunk, pl.ds(rlo, RPS)])


@functools.lru_cache(maxsize=None)
def _make_agg(C, F, NBt, table_C, split):
    mesh = plsc.VectorSubcoreMesh(core_axis_name="c", subcore_axis_name="s")
    return pl.kernel(
        functools.partial(_agg_body, C, F, NBt, table_C, split),
        out_type=jax.ShapeDtypeStruct((C, NR, F), jnp.float32),
        mesh=mesh,
        scratch_types=[
            pltpu.VMEM((NBt, EB), jnp.int32)] +
        [pltpu.VMEM((EBH,), jnp.int32) for _ in range(8)] +
        [pltpu.VMEM((EBH, F), jnp.float32) for _ in range(4)] +
        [
            pltpu.VMEM((16, F), jnp.float32),
            pltpu.VMEM_SHARED((NR, F), jnp.float32),
        ] + [pltpu.SemaphoreType.DMA] * 8,
    )


def _deg_body(NBt, pk, ones_in, zeros_in, out,
              pk_v, ida, idb, ones_v, zbuf, acc, sem_sa, sem_sb):
    cid = lax.axis_index("c")
    sid = lax.axis_index("s")
    rlo = sid * RPS
    nb = 2 * NBt // NC
    jbase = cid * nb
    pltpu.sync_copy(pk.at[sid], pk_v)
    pltpu.sync_copy(ones_in, ones_v)
    pltpu.sync_copy(zeros_in, zbuf)

    def unpack(j, di):
        for k in range(EBH // 16):
            di[pl.ds(k * 16, 16)] = lax.bitwise_and(
                pk_v[j // 2, pl.ds((j % 2) * EBH + k * 16, 16)], 0xFFFF)

    for z in range(RPS // 16):
        pltpu.async_copy(zbuf, acc.at[pl.ds(rlo + z * 16, 16)], sem_sa)
    for z in range(RPS // 16):
        pltpu.make_async_copy(
            zbuf, acc.at[pl.ds(rlo + z * 16, 16)], sem_sa).wait()
    plsc.subcore_barrier()
    unpack(jbase, ida)
    pltpu.async_copy(ones_v, acc.at[ida], sem_sa, add=True)
    unpack(jbase + 1, idb)
    pltpu.async_copy(ones_v, acc.at[idb], sem_sb, add=True)

    def pair(p, carry):
        j0 = jbase + 2 * p
        pltpu.make_async_copy(ones_v, acc.at[ida], sem_sa).wait()
        unpack(j0 + 2, ida)
        pltpu.async_copy(ones_v, acc.at[ida], sem_sa, add=True)
        pltpu.make_async_copy(ones_v, acc.at[idb], sem_sb).wait()
        unpack(j0 + 3, idb)
        pltpu.async_copy(ones_v, acc.at[idb], sem_sb, add=True)
        return carry

    lax.fori_loop(0, nb // 2 - 1, pair, 0)
    pltpu.make_async_copy(ones_v, acc.at[ida], sem_sa).wait()
    pltpu.make_async_copy(ones_v, acc.at[idb], sem_sb).wait()
    plsc.subcore_barrier()
    pltpu.sync_copy(acc.at[pl.ds(rlo, RPS)], out.at[cid, pl.ds(rlo, RPS)])


@functools.lru_cache(maxsize=None)
def _make_deg(NBt):
    mesh = plsc.VectorSubcoreMesh(core_axis_name="c", subcore_axis_name="s")
    return pl.kernel(
        functools.partial(_deg_body, NBt),
        out_type=jax.ShapeDtypeStruct((NC, NR, DW), jnp.float32),
        mesh=mesh,
        scratch_types=[
            pltpu.VMEM((NBt, EB), jnp.int32),
            pltpu.VMEM((EBH,), jnp.int32),
            pltpu.VMEM((EBH,), jnp.int32),
            pltpu.VMEM((EBH, DW), jnp.float32),
            pltpu.VMEM((16, DW), jnp.float32),
            pltpu.VMEM_SHARED((NR, DW), jnp.float32),
            pltpu.SemaphoreType.DMA,
            pltpu.SemaphoreType.DMA,
        ],
    )


def _mm_u1(x_pad, W, C_out, F_out):
    d_in = x_pad.shape[1]

    def body(x_ref, w_ref, out_ref):
        res = jnp.dot(x_ref[...], w_ref[...],
                      preferred_element_type=jnp.float32)
        for c2 in range(C_out):
            out_ref[c2] = res[:, c2 * F_out:(c2 + 1) * F_out]

    return pl.pallas_call(
        body,
        grid=(NR // BR,),
        in_specs=[
            pl.BlockSpec((BR, d_in), lambda i: (i, 0)),
            pl.BlockSpec((d_in, C_out * F_out), lambda i: (0, 0)),
        ],
        out_specs=pl.BlockSpec((C_out, BR, F_out), lambda i: (0, i, 0)),
        out_shape=jax.ShapeDtypeStruct((C_out, NR, F_out), jnp.float32),
    )(x_pad, W)


def _scale_first(u1, degs, C):
    def body(u_ref, d_ref, t_ref, dv_ref):
        deg = d_ref[0, :, 0:1] + d_ref[1, :, 0:1] + 1.0
        dv = 1.0 / jnp.sqrt(deg)
        for c in range(C):
            t_ref[c] = u_ref[c] * dv
        dv_ref[...] = dv

    return pl.pallas_call(
        body,
        grid=(NR // BR,),
        in_specs=[
            pl.BlockSpec((C, BR, 128), lambda i: (0, i, 0)),
            pl.BlockSpec((2, BR, DW), lambda i: (0, i, 0)),
        ],
        out_specs=[
            pl.BlockSpec((C, BR, 128), lambda i: (0, i, 0)),
            pl.BlockSpec((BR, 1), lambda i: (i, 0)),
        ],
        out_shape=[
            jax.ShapeDtypeStruct((C, NR, 128), jnp.float32),
            jax.ShapeDtypeStruct((NR, 1), jnp.float32),
        ],
    )(u1, degs)


def _mm_mid(agg, tbl, dinv, b_prev, W, C_in, F_in, C_out, F_out,
            sum_in=False):
    d_out = W.shape[1]
    tc = 1 if sum_in else C_in
    w_r = W.reshape(tc, F_in, d_out)
    b_r = b_prev.reshape(tc, 1, F_in)

    def body(a_ref, t_ref, dv_ref, b_ref, w_ref, out_ref):
        dv = dv_ref[...]
        if sum_in:
            asum = t_ref[0]
            for c in range(C_in):
                asum = asum + a_ref[c]
            xc = jnp.maximum(asum * dv + b_ref[0], 0.0)
            acc = jnp.dot(xc, w_ref[0], preferred_element_type=jnp.float32)
        else:
            acc = jnp.zeros((BR, d_out), jnp.float32)
            for c in range(C_in):
                xc = jnp.maximum((a_ref[c] + t_ref[c]) * dv + b_ref[c], 0.0)
                acc = acc + jnp.dot(xc, w_ref[c],
                                    preferred_element_type=jnp.float32)
        res = acc * dv
        for c2 in range(C_out):
            out_ref[c2] = res[:, c2 * F_out:(c2 + 1) * F_out]

    return pl.pallas_call(
        body,
        grid=(NR // BR,),
        in_specs=[
            pl.BlockSpec((C_in, BR, F_in), lambda i: (0, i, 0)),
            pl.BlockSpec((tc, BR, F_in), lambda i: (0, i, 0)),
            pl.BlockSpec((BR, 1), lambda i: (i, 0)),
            pl.BlockSpec((tc, 1, F_in), lambda i: (0, 0, 0)),
            pl.BlockSpec((tc, F_in, d_out), lambda i: (0, 0, 0)),
        ],
        out_specs=pl.BlockSpec((C_out, BR, F_out), lambda i: (0, i, 0)),
        out_shape=jax.ShapeDtypeStruct((C_out, NR, F_out), jnp.float32),
    )(agg, tbl, dinv, b_r, w_r)


def _mm_last(agg, tbl, dinv, b4):
    b_r = b4.reshape(2, 1, 128)
    blk = 400

    def body(a_ref, t_ref, dv_ref, b_ref, out_ref):
        dv = dv_ref[...]
        for c in range(2):
            out_ref[:, c * 128:(c + 1) * 128] = \
                (a_ref[c] + t_ref[c]) * dv + b_ref[c]

    return pl.pallas_call(
        body,
        grid=(N // blk,),
        in_specs=[
            pl.BlockSpec((2, blk, 128), lambda i: (0, i, 0)),
            pl.BlockSpec((2, blk, 128), lambda i: (0, i, 0)),
            pl.BlockSpec((blk, 1), lambda i: (i, 0)),
            pl.BlockSpec((2, 1, 128), lambda i: (0, 0, 0)),
        ],
        out_specs=pl.BlockSpec((blk, 256), lambda i: (i, 0)),
        out_shape=jax.ShapeDtypeStruct((N, 256), jnp.float32),
    )(agg, tbl, dinv, b_r)


def kernel(x, edge_index, W1, b1, W2, b2, W3, b3, W4, b4):
    E = edge_index.shape[1]
    src_f = edge_index[0]
    dst_f = edge_index[1]
    nbt = -(-E // (NS * EB))
    nbt = (nbt + 3) // 4 * 4
    pad = NS * nbt * EB - E
    dummy = N + jnp.arange(pad, dtype=jnp.int32) % (NR - N)
    flat = jnp.concatenate(
        [src_f * 65536 + dst_f, dummy * 65536 + dummy])
    packed = flat.reshape(NS, nbt, EB)

    zeros128 = jnp.zeros((16, 128), jnp.float32)
    ones128 = jnp.ones((EBH, DW), jnp.float32)

    x_pad = jnp.concatenate(
        [x, jnp.zeros((NR - N, IN_DIM), jnp.float32)], axis=0)

    degs = _make_deg(nbt)(packed, ones128, zeros128)
    u1 = _mm_u1(x_pad, W1, 4, 128)
    t1, dinv = _scale_first(u1, degs, 4)
    a1 = _make_agg(4, 128, nbt, 4, False)(
        t1.reshape(4 * NR, 128), packed, zeros128)
    t2 = _mm_mid(a1, t1, dinv, b1, W2, 4, 128, 1, 128)
    a2 = _make_agg(2, 128, nbt, 1, True)(
        t2.reshape(NR, 128), packed, zeros128)
    t3 = _mm_mid(a2, t2, dinv, b2, W3, 2, 128, 4, 128, sum_in=True)
    a3 = _make_agg(4, 128, nbt, 4, False)(
        t3.reshape(4 * NR, 128), packed, zeros128)
    t4 = _mm_mid(a3, t3, dinv, b3, W4, 4, 128, 2, 128)
    a4 = _make_agg(2, 128, nbt, 2, False)(
        t4.reshape(2 * NR, 128), packed, zeros128)
    return _mm_last(a4, t4, dinv, b4)


# speedup vs baseline: 14.5915x; 1.0283x over previous
import functools

import jax
import jax.numpy as jnp
from jax import lax
from jax.experimental import pallas as pl
from jax.experimental.pallas import tpu as pltpu
from jax.experimental.pallas import tpu_sc as plsc

N = 10000
IN_DIM = 256

NR = 10240
NC = 2
NS = 16
EB = 128
EBH = 64
RPS = NR // NS
DW = 128
BR = 1024


def _agg_body(C, F, NBt, table_C, split, table, pk, zeros_in, out,
              pk_v, is0, id0, is1, id1, is2, id2, is3, id3,
              buf0, buf1, buf2, buf3, zbuf, acc,
              sg0, sg1, sg2, sg3, ss0, ss1, ss2, ss3):
    cid = lax.axis_index("c")
    sid = lax.axis_index("s")
    rlo = sid * RPS
    cpc = C // NC
    nbh = 2 * NBt
    nb = nbh // NC if split else nbh
    iss = [is0, is1, is2, is3]
    ids = [id0, id1, id2, id3]
    bufs = [buf0, buf1, buf2, buf3]
    sgs = [sg0, sg1, sg2, sg3]
    sss = [ss0, ss1, ss2, ss3]
    pltpu.sync_copy(pk.at[sid], pk_v)
    pltpu.sync_copy(zeros_in, zbuf)

    def unpack(j, off, si, di):
        for k in range(EBH // 16):
            v = pk_v[j // 2, pl.ds((j % 2) * EBH + k * 16, 16)]
            si[pl.ds(k * 16, 16)] = lax.shift_right_logical(v, 16) + off
            di[pl.ds(k * 16, 16)] = lax.bitwise_and(v, 0xFFFF)

    for local in range(cpc):
        chunk = cid * cpc + local
        off = chunk * NR if table_C == C else 0
        jbase = cid * nb if split else 0
        for z in range(RPS // 16):
            pltpu.async_copy(zbuf, acc.at[pl.ds(rlo + z * 16, 16)], sg0)
        for z in range(RPS // 16):
            pltpu.make_async_copy(
                zbuf, acc.at[pl.ds(rlo + z * 16, 16)], sg0).wait()
        plsc.subcore_barrier()
        for t in range(4):
            unpack(jbase + t, off, iss[t], ids[t])
            pltpu.async_copy(table.at[iss[t]], bufs[t], sgs[t])

        def grp(p, carry):
            j0 = jbase + 4 * p
            scs = []
            for t in range(4):
                pltpu.make_async_copy(table.at[iss[t]], bufs[t], sgs[t]).wait()
                scs.append(pltpu.async_copy(
                    bufs[t], acc.at[ids[t]], sss[t], add=True))
            for t in range(4):
                scs[t].wait()
                unpack(j0 + 4 + t, off, iss[t], ids[t])
                pltpu.async_copy(table.at[iss[t]], bufs[t], sgs[t])
            return carry

        lax.fori_loop(0, nb // 4 - 1, grp, 0)
        scs = []
        for t in range(4):
            pltpu.make_async_copy(table.at[iss[t]], bufs[t], sgs[t]).wait()
            scs.append(pltpu.async_copy(
                bufs[t], acc.at[ids[t]], sss[t], add=True))
        for t in range(4):
            scs[t].wait()
        plsc.subcore_barrier()
        pltpu.sync_copy(acc.at[pl.ds(rlo, RPS)],
                        out.at[chunk, pl.ds(rlo, RPS)])


@functools.lru_cache(maxsize=None)
def _make_agg(C, F, NBt, table_C, split):
    mesh = plsc.VectorSubcoreMesh(core_axis_name="c", subcore_axis_name="s")
    return pl.kernel(
        functools.partial(_agg_body, C, F, NBt, table_C, split),
        out_type=jax.ShapeDtypeStruct((C, NR, F), jnp.float32),
        mesh=mesh,
        scratch_types=[
            pltpu.VMEM((NBt, EB), jnp.int32)] +
        [pltpu.VMEM((EBH,), jnp.int32) for _ in range(8)] +
        [pltpu.VMEM((EBH, F), jnp.float32) for _ in range(4)] +
        [
            pltpu.VMEM((16, F), jnp.float32),
            pltpu.VMEM_SHARED((NR, F), jnp.float32),
        ] + [pltpu.SemaphoreType.DMA] * 8,
    )


def _deg_body(NBt, pk, ones_in, zeros_in, out,
              pk_v, ida, idb, ones_v, zbuf, acc, sem_sa, sem_sb):
    cid = lax.axis_index("c")
    sid = lax.axis_index("s")
    rlo = sid * RPS
    nb = 2 * NBt // NC
    jbase = cid * nb
    pltpu.sync_copy(pk.at[sid], pk_v)
    pltpu.sync_copy(ones_in, ones_v)
    pltpu.sync_copy(zeros_in, zbuf)

    def unpack(j, di):
        for k in range(EBH // 16):
            di[pl.ds(k * 16, 16)] = lax.bitwise_and(
                pk_v[j // 2, pl.ds((j % 2) * EBH + k * 16, 16)], 0xFFFF)

    for z in range(RPS // 16):
        pltpu.async_copy(zbuf, acc.at[pl.ds(rlo + z * 16, 16)], sem_sa)
    for z in range(RPS // 16):
        pltpu.make_async_copy(
            zbuf, acc.at[pl.ds(rlo + z * 16, 16)], sem_sa).wait()
    plsc.subcore_barrier()
    unpack(jbase, ida)
    pltpu.async_copy(ones_v, acc.at[ida], sem_sa, add=True)
    unpack(jbase + 1, idb)
    pltpu.async_copy(ones_v, acc.at[idb], sem_sb, add=True)

    def pair(p, carry):
        j0 = jbase + 2 * p
        pltpu.make_async_copy(ones_v, acc.at[ida], sem_sa).wait()
        unpack(j0 + 2, ida)
        pltpu.async_copy(ones_v, acc.at[ida], sem_sa, add=True)
        pltpu.make_async_copy(ones_v, acc.at[idb], sem_sb).wait()
        unpack(j0 + 3, idb)
        pltpu.async_copy(ones_v, acc.at[idb], sem_sb, add=True)
        return carry

    lax.fori_loop(0, nb // 2 - 1, pair, 0)
    pltpu.make_async_copy(ones_v, acc.at[ida], sem_sa).wait()
    pltpu.make_async_copy(ones_v, acc.at[idb], sem_sb).wait()
    plsc.subcore_barrier()
    pltpu.sync_copy(acc.at[pl.ds(rlo, RPS)], out.at[cid, pl.ds(rlo, RPS)])


@functools.lru_cache(maxsize=None)
def _make_deg(NBt):
    mesh = plsc.VectorSubcoreMesh(core_axis_name="c", subcore_axis_name="s")
    return pl.kernel(
        functools.partial(_deg_body, NBt),
        out_type=jax.ShapeDtypeStruct((NC, NR, DW), jnp.float32),
        mesh=mesh,
        scratch_types=[
            pltpu.VMEM((NBt, EB), jnp.int32),
            pltpu.VMEM((EBH,), jnp.int32),
            pltpu.VMEM((EBH,), jnp.int32),
            pltpu.VMEM((EBH, DW), jnp.float32),
            pltpu.VMEM((16, DW), jnp.float32),
            pltpu.VMEM_SHARED((NR, DW), jnp.float32),
            pltpu.SemaphoreType.DMA,
            pltpu.SemaphoreType.DMA,
        ],
    )


def _mm_u1(x_pad, W, C_out, F_out):
    d_in = x_pad.shape[1]

    def body(x_ref, w_ref, out_ref):
        res = jnp.dot(x_ref[...], w_ref[...],
                      preferred_element_type=jnp.float32)
        for c2 in range(C_out):
            out_ref[c2] = res[:, c2 * F_out:(c2 + 1) * F_out]

    return pl.pallas_call(
        body,
        grid=(NR // BR,),
        in_specs=[
            pl.BlockSpec((BR, d_in), lambda i: (i, 0)),
            pl.BlockSpec((d_in, C_out * F_out), lambda i: (0, 0)),
        ],
        out_specs=pl.BlockSpec((C_out, BR, F_out), lambda i: (0, i, 0)),
        out_shape=jax.ShapeDtypeStruct((C_out, NR, F_out), jnp.float32),
    )(x_pad, W)


def _scale_first(u1, degs, C):
    def body(u_ref, d_ref, t_ref, dv_ref):
        deg = d_ref[0, :, 0:1] + d_ref[1, :, 0:1] + 1.0
        dv = 1.0 / jnp.sqrt(deg)
        for c in range(C):
            t_ref[c] = u_ref[c] * dv
        dv_ref[...] = dv

    return pl.pallas_call(
        body,
        grid=(NR // BR,),
        in_specs=[
            pl.BlockSpec((C, BR, 128), lambda i: (0, i, 0)),
            pl.BlockSpec((2, BR, DW), lambda i: (0, i, 0)),
        ],
        out_specs=[
            pl.BlockSpec((C, BR, 128), lambda i: (0, i, 0)),
            pl.BlockSpec((BR, 1), lambda i: (i, 0)),
        ],
        out_shape=[
            jax.ShapeDtypeStruct((C, NR, 128), jnp.float32),
            jax.ShapeDtypeStruct((NR, 1), jnp.float32),
        ],
    )(u1, degs)


def _mm_mid(agg, tbl, dinv, b_prev, W, C_in, F_in, C_out, F_out,
            sum_in=False):
    d_out = W.shape[1]
    tc = 1 if sum_in else C_in
    w_r = W.reshape(tc, F_in, d_out)
    b_r = b_prev.reshape(tc, 1, F_in)

    def body(a_ref, t_ref, dv_ref, b_ref, w_ref, out_ref):
        dv = dv_ref[...]
        if sum_in:
            asum = t_ref[0]
            for c in range(C_in):
                asum = asum + a_ref[c]
            xc = jnp.maximum(asum * dv + b_ref[0], 0.0)
            acc = jnp.dot(xc, w_ref[0], preferred_element_type=jnp.float32)
        else:
            acc = jnp.zeros((BR, d_out), jnp.float32)
            for c in range(C_in):
                xc = jnp.maximum((a_ref[c] + t_ref[c]) * dv + b_ref[c], 0.0)
                acc = acc + jnp.dot(xc, w_ref[c],
                                    preferred_element_type=jnp.float32)
        res = acc * dv
        for c2 in range(C_out):
            out_ref[c2] = res[:, c2 * F_out:(c2 + 1) * F_out]

    return pl.pallas_call(
        body,
        grid=(NR // BR,),
        in_specs=[
            pl.BlockSpec((C_in, BR, F_in), lambda i: (0, i, 0)),
            pl.BlockSpec((tc, BR, F_in), lambda i: (0, i, 0)),
            pl.BlockSpec((BR, 1), lambda i: (i, 0)),
            pl.BlockSpec((tc, 1, F_in), lambda i: (0, 0, 0)),
            pl.BlockSpec((tc, F_in, d_out), lambda i: (0, 0, 0)),
        ],
        out_specs=pl.BlockSpec((C_out, BR, F_out), lambda i: (0, i, 0)),
        out_shape=jax.ShapeDtypeStruct((C_out, NR, F_out), jnp.float32),
    )(agg, tbl, dinv, b_r, w_r)


def _mm_last(agg, tbl, dinv, b4):
    b_r = b4.reshape(2, 1, 128)
    blk = 400

    def body(a_ref, t_ref, dv_ref, b_ref, out_ref):
        dv = dv_ref[...]
        for c in range(2):
            out_ref[:, c * 128:(c + 1) * 128] = \
                (a_ref[c] + t_ref[c]) * dv + b_ref[c]

    return pl.pallas_call(
        body,
        grid=(N // blk,),
        in_specs=[
            pl.BlockSpec((2, blk, 128), lambda i: (0, i, 0)),
            pl.BlockSpec((2, blk, 128), lambda i: (0, i, 0)),
            pl.BlockSpec((blk, 1), lambda i: (i, 0)),
            pl.BlockSpec((2, 1, 128), lambda i: (0, 0, 0)),
        ],
        out_specs=pl.BlockSpec((blk, 256), lambda i: (i, 0)),
        out_shape=jax.ShapeDtypeStruct((N, 256), jnp.float32),
    )(agg, tbl, dinv, b_r)


def kernel(x, edge_index, W1, b1, W2, b2, W3, b3, W4, b4):
    E = edge_index.shape[1]
    src_f = edge_index[0]
    dst_f = edge_index[1]
    nbt = -(-E // (NS * EB))
    nbt = (nbt + 3) // 4 * 4
    pad = NS * nbt * EB - E
    dummy = N + jnp.arange(pad, dtype=jnp.int32) % (NR - N)
    flat = jnp.concatenate(
        [src_f * 65536 + dst_f, dummy * 65536 + dummy])
    packed = flat.reshape(NS, nbt, EB)

    zeros128 = jnp.zeros((16, 128), jnp.float32)
    ones128 = jnp.ones((EBH, DW), jnp.float32)

    x_pad = jnp.concatenate(
        [x, jnp.zeros((NR - N, IN_DIM), jnp.float32)], axis=0)

    degs = _make_deg(nbt)(packed, ones128, zeros128)
    u1 = _mm_u1(x_pad, W1, 4, 128)
    t1, dinv = _scale_first(u1, degs, 4)
    a1 = _make_agg(4, 128, nbt, 4, False)(
        t1.reshape(4 * NR, 128), packed, zeros128)
    t2 = _mm_mid(a1, t1, dinv, b1, W2, 4, 128, 1, 128)
    a2 = _make_agg(2, 128, nbt, 1, True)(
        t2.reshape(NR, 128), packed, zeros128)
    t3 = _mm_mid(a2, t2, dinv, b2, W3, 2, 128, 4, 128, sum_in=True)
    a3 = _make_agg(4, 128, nbt, 4, False)(
        t3.reshape(4 * NR, 128), packed, zeros128)
    t4 = _mm_mid(a3, t3, dinv, b3, W4, 4, 128, 2, 128)
    a4 = _make_agg(2, 128, nbt, 2, False)(
        t4.reshape(2 * NR, 128), packed, zeros128)
    return _mm_last(a4, t4, dinv, b4)


# speedup vs baseline: 14.6786x; 1.0060x over previous
import functools

import jax
import jax.numpy as jnp
from jax import lax
from jax.experimental import pallas as pl
from jax.experimental.pallas import tpu as pltpu
from jax.experimental.pallas import tpu_sc as plsc

N = 10000
IN_DIM = 256

NR = 10240
NC = 2
NS = 16
EB = 128
EBH = 64
RPS = NR // NS
DW = 128
BR = 2048


def _agg_body(C, F, NBt, table_C, split, table, pk, zeros_in, out,
              pk_v, is0, id0, is1, id1, is2, id2, is3, id3,
              buf0, buf1, buf2, buf3, zbuf, acc,
              sg0, sg1, sg2, sg3, ss0, ss1, ss2, ss3):
    cid = lax.axis_index("c")
    sid = lax.axis_index("s")
    rlo = sid * RPS
    cpc = C // NC
    nbh = 2 * NBt
    nb = nbh // NC if split else nbh
    iss = [is0, is1, is2, is3]
    ids = [id0, id1, id2, id3]
    bufs = [buf0, buf1, buf2, buf3]
    sgs = [sg0, sg1, sg2, sg3]
    sss = [ss0, ss1, ss2, ss3]
    pltpu.sync_copy(pk.at[sid], pk_v)
    pltpu.sync_copy(zeros_in, zbuf)

    def unpack(j, off, si, di):
        for k in range(EBH // 16):
            v = pk_v[j // 2, pl.ds((j % 2) * EBH + k * 16, 16)]
            si[pl.ds(k * 16, 16)] = lax.shift_right_logical(v, 16) + off
            di[pl.ds(k * 16, 16)] = lax.bitwise_and(v, 0xFFFF)

    for local in range(cpc):
        chunk = cid * cpc + local
        off = chunk * NR if table_C == C else 0
        jbase = cid * nb if split else 0
        for z in range(RPS // 16):
            pltpu.async_copy(zbuf, acc.at[pl.ds(rlo + z * 16, 16)], sg0)
        for z in range(RPS // 16):
            pltpu.make_async_copy(
                zbuf, acc.at[pl.ds(rlo + z * 16, 16)], sg0).wait()
        plsc.subcore_barrier()
        for t in range(4):
            unpack(jbase + t, off, iss[t], ids[t])
            pltpu.async_copy(table.at[iss[t]], bufs[t], sgs[t])

        def grp(p, carry):
            j0 = jbase + 4 * p
            scs = []
            for t in range(4):
                pltpu.make_async_copy(table.at[iss[t]], bufs[t], sgs[t]).wait()
                scs.append(pltpu.async_copy(
                    bufs[t], acc.at[ids[t]], sss[t], add=True))
            for t in range(4):
                scs[t].wait()
                unpack(j0 + 4 + t, off, iss[t], ids[t])
                pltpu.async_copy(table.at[iss[t]], bufs[t], sgs[t])
            return carry

        lax.fori_loop(0, nb // 4 - 1, grp, 0)
        scs = []
        for t in range(4):
            pltpu.make_async_copy(table.at[iss[t]], bufs[t], sgs[t]).wait()
            scs.append(pltpu.async_copy(
                bufs[t], acc.at[ids[t]], sss[t], add=True))
        for t in range(4):
            scs[t].wait()
        plsc.subcore_barrier()
        pltpu.sync_copy(acc.at[pl.ds(rlo, RPS)],
                        out.at[chunk, pl.ds(rlo, RPS)])


@functools.lru_cache(maxsize=None)
def _make_agg(C, F, NBt, table_C, split):
    mesh = plsc.VectorSubcoreMesh(core_axis_name="c", subcore_axis_name="s")
    return pl.kernel(
        functools.partial(_agg_body, C, F, NBt, table_C, split),
        out_type=jax.ShapeDtypeStruct((C, NR, F), jnp.float32),
        mesh=mesh,
        scratch_types=[
            pltpu.VMEM((NBt, EB), jnp.int32)] +
        [pltpu.VMEM((EBH,), jnp.int32) for _ in range(8)] +
        [pltpu.VMEM((EBH, F), jnp.float32) for _ in range(4)] +
        [
            pltpu.VMEM((16, F), jnp.float32),
            pltpu.VMEM_SHARED((NR, F), jnp.float32),
        ] + [pltpu.SemaphoreType.DMA] * 8,
    )


def _deg_body(NBt, pk, ones_in, zeros_in, out,
              pk_v, ida, idb, ones_v, zbuf, acc, sem_sa, sem_sb):
    cid = lax.axis_index("c")
    sid = lax.axis_index("s")
    rlo = sid * RPS
    nb = 2 * NBt // NC
    jbase = cid * nb
    pltpu.sync_copy(pk.at[sid], pk_v)
    pltpu.sync_copy(ones_in, ones_v)
    pltpu.sync_copy(zeros_in, zbuf)

    def unpack(j, di):
        for k in range(EBH // 16):
            di[pl.ds(k * 16, 16)] = lax.bitwise_and(
                pk_v[j // 2, pl.ds((j % 2) * EBH + k * 16, 16)], 0xFFFF)

    for z in range(RPS // 16):
        pltpu.async_copy(zbuf, acc.at[pl.ds(rlo + z * 16, 16)], sem_sa)
    for z in range(RPS // 16):
        pltpu.make_async_copy(
            zbuf, acc.at[pl.ds(rlo + z * 16, 16)], sem_sa).wait()
    plsc.subcore_barrier()
    unpack(jbase, ida)
    pltpu.async_copy(ones_v, acc.at[ida], sem_sa, add=True)
    unpack(jbase + 1, idb)
    pltpu.async_copy(ones_v, acc.at[idb], sem_sb, add=True)

    def pair(p, carry):
        j0 = jbase + 2 * p
        pltpu.make_async_copy(ones_v, acc.at[ida], sem_sa).wait()
        unpack(j0 + 2, ida)
        pltpu.async_copy(ones_v, acc.at[ida], sem_sa, add=True)
        pltpu.make_async_copy(ones_v, acc.at[idb], sem_sb).wait()
        unpack(j0 + 3, idb)
        pltpu.async_copy(ones_v, acc.at[idb], sem_sb, add=True)
        return carry

    lax.fori_loop(0, nb // 2 - 1, pair, 0)
    pltpu.make_async_copy(ones_v, acc.at[ida], sem_sa).wait()
    pltpu.make_async_copy(ones_v, acc.at[idb], sem_sb).wait()
    plsc.subcore_barrier()
    pltpu.sync_copy(acc.at[pl.ds(rlo, RPS)], out.at[cid, pl.ds(rlo, RPS)])


@functools.lru_cache(maxsize=None)
def _make_deg(NBt):
    mesh = plsc.VectorSubcoreMesh(core_axis_name="c", subcore_axis_name="s")
    return pl.kernel(
        functools.partial(_deg_body, NBt),
        out_type=jax.ShapeDtypeStruct((NC, NR, DW), jnp.float32),
        mesh=mesh,
        scratch_types=[
            pltpu.VMEM((NBt, EB), jnp.int32),
            pltpu.VMEM((EBH,), jnp.int32),
            pltpu.VMEM((EBH,), jnp.int32),
            pltpu.VMEM((EBH, DW), jnp.float32),
            pltpu.VMEM((16, DW), jnp.float32),
            pltpu.VMEM_SHARED((NR, DW), jnp.float32),
            pltpu.SemaphoreType.DMA,
            pltpu.SemaphoreType.DMA,
        ],
    )


def _mm_u1(x_pad, W, C_out, F_out):
    d_in = x_pad.shape[1]

    def body(x_ref, w_ref, out_ref):
        res = jnp.dot(x_ref[...], w_ref[...],
                      preferred_element_type=jnp.float32)
        for c2 in range(C_out):
            out_ref[c2] = res[:, c2 * F_out:(c2 + 1) * F_out]

    return pl.pallas_call(
        body,
        grid=(NR // BR,),
        in_specs=[
            pl.BlockSpec((BR, d_in), lambda i: (i, 0)),
            pl.BlockSpec((d_in, C_out * F_out), lambda i: (0, 0)),
        ],
        out_specs=pl.BlockSpec((C_out, BR, F_out), lambda i: (0, i, 0)),
        out_shape=jax.ShapeDtypeStruct((C_out, NR, F_out), jnp.float32),
    )(x_pad, W)


def _scale_first(u1, degs, C):
    def body(u_ref, d_ref, t_ref, dv_ref):
        deg = d_ref[0, :, 0:1] + d_ref[1, :, 0:1] + 1.0
        dv = 1.0 / jnp.sqrt(deg)
        for c in range(C):
            t_ref[c] = u_ref[c] * dv
        dv_ref[...] = dv

    return pl.pallas_call(
        body,
        grid=(NR // BR,),
        in_specs=[
            pl.BlockSpec((C, BR, 128), lambda i: (0, i, 0)),
            pl.BlockSpec((2, BR, DW), lambda i: (0, i, 0)),
        ],
        out_specs=[
            pl.BlockSpec((C, BR, 128), lambda i: (0, i, 0)),
            pl.BlockSpec((BR, 1), lambda i: (i, 0)),
        ],
        out_shape=[
            jax.ShapeDtypeStruct((C, NR, 128), jnp.float32),
            jax.ShapeDtypeStruct((NR, 1), jnp.float32),
        ],
    )(u1, degs)


def _mm_mid(agg, tbl, dinv, b_prev, W, C_in, F_in, C_out, F_out,
            sum_in=False):
    d_out = W.shape[1]
    tc = 1 if sum_in else C_in
    w_r = W.reshape(tc, F_in, d_out)
    b_r = b_prev.reshape(tc, 1, F_in)

    def body(a_ref, t_ref, dv_ref, b_ref, w_ref, out_ref):
        dv = dv_ref[...]
        if sum_in:
            asum = t_ref[0]
            for c in range(C_in):
                asum = asum + a_ref[c]
            xc = jnp.maximum(asum * dv + b_ref[0], 0.0)
            acc = jnp.dot(xc, w_ref[0], preferred_element_type=jnp.float32)
        else:
            acc = jnp.zeros((BR, d_out), jnp.float32)
            for c in range(C_in):
                xc = jnp.maximum((a_ref[c] + t_ref[c]) * dv + b_ref[c], 0.0)
                acc = acc + jnp.dot(xc, w_ref[c],
                                    preferred_element_type=jnp.float32)
        res = acc * dv
        for c2 in range(C_out):
            out_ref[c2] = res[:, c2 * F_out:(c2 + 1) * F_out]

    return pl.pallas_call(
        body,
        grid=(NR // BR,),
        in_specs=[
            pl.BlockSpec((C_in, BR, F_in), lambda i: (0, i, 0)),
            pl.BlockSpec((tc, BR, F_in), lambda i: (0, i, 0)),
            pl.BlockSpec((BR, 1), lambda i: (i, 0)),
            pl.BlockSpec((tc, 1, F_in), lambda i: (0, 0, 0)),
            pl.BlockSpec((tc, F_in, d_out), lambda i: (0, 0, 0)),
        ],
        out_specs=pl.BlockSpec((C_out, BR, F_out), lambda i: (0, i, 0)),
        out_shape=jax.ShapeDtypeStruct((C_out, NR, F_out), jnp.float32),
    )(agg, tbl, dinv, b_r, w_r)


def _mm_last(agg, tbl, dinv, b4):
    b_r = b4.reshape(2, 1, 128)
    blk = 400

    def body(a_ref, t_ref, dv_ref, b_ref, out_ref):
        dv = dv_ref[...]
        for c in range(2):
            out_ref[:, c * 128:(c + 1) * 128] = \
                (a_ref[c] + t_ref[c]) * dv + b_ref[c]

    return pl.pallas_call(
        body,
        grid=(N // blk,),
        in_specs=[
            pl.BlockSpec((2, blk, 128), lambda i: (0, i, 0)),
            pl.BlockSpec((2, blk, 128), lambda i: (0, i, 0)),
            pl.BlockSpec((blk, 1), lambda i: (i, 0)),
            pl.BlockSpec((2, 1, 128), lambda i: (0, 0, 0)),
        ],
        out_specs=pl.BlockSpec((blk, 256), lambda i: (i, 0)),
        out_shape=jax.ShapeDtypeStruct((N, 256), jnp.float32),
    )(agg, tbl, dinv, b_r)


def kernel(x, edge_index, W1, b1, W2, b2, W3, b3, W4, b4):
    E = edge_index.shape[1]
    src_f = edge_index[0]
    dst_f = edge_index[1]
    nbt = -(-E // (NS * EB))
    nbt = (nbt + 3) // 4 * 4
    pad = NS * nbt * EB - E
    dummy = N + jnp.arange(pad, dtype=jnp.int32) % (NR - N)
    flat = jnp.concatenate(
        [src_f * 65536 + dst_f, dummy * 65536 + dummy])
    packed = flat.reshape(NS, nbt, EB)

    zeros128 = jnp.zeros((16, 128), jnp.float32)
    ones128 = jnp.ones((EBH, DW), jnp.float32)

    x_pad = jnp.concatenate(
        [x, jnp.zeros((NR - N, IN_DIM), jnp.float32)], axis=0)

    degs = _make_deg(nbt)(packed, ones128, zeros128)
    u1 = _mm_u1(x_pad, W1, 4, 128)
    t1, dinv = _scale_first(u1, degs, 4)
    a1 = _make_agg(4, 128, nbt, 4, False)(
        t1.reshape(4 * NR, 128), packed, zeros128)
    t2 = _mm_mid(a1, t1, dinv, b1, W2, 4, 128, 1, 128)
    a2 = _make_agg(2, 128, nbt, 1, True)(
        t2.reshape(NR, 128), packed, zeros128)
    t3 = _mm_mid(a2, t2, dinv, b2, W3, 2, 128, 4, 128, sum_in=True)
    a3 = _make_agg(4, 128, nbt, 4, False)(
        t3.reshape(4 * NR, 128), packed, zeros128)
    t4 = _mm_mid(a3, t3, dinv, b3, W4, 4, 128, 2, 128)
    a4 = _make_agg(2, 128, nbt, 2, False)(
        t4.reshape(2 * NR, 128), packed, zeros128)
    return _mm_last(a4, t4, dinv, b4)
